# Initial kernel scaffold; baseline (speedup 1.0000x reference)
import jax
import jax.numpy as jnp
from jax import lax
from jax.experimental import pallas as pl
from jax.experimental.pallas import tpu as pltpu
from jax.experimental.pallas import tpu_sc as plsc

N = 100000
E = 1600000
NT = 16
NC = 2
CH = 2000
ROWS = 16
COLS = 125
NCHUNK = E // NT // CH
NPAD = 100352
TILE_N = NPAD // NT

_mesh = plsc.VectorSubcoreMesh(core_axis_name="c", subcore_axis_name="s")


def _lrelu(h):
    return jnp.where(h >= 0, h, 0.1 * h)


def _bn(a, g, b, eps=1e-5):
    m = jnp.mean(a, axis=0)
    v = jnp.mean(a * a, axis=0) - m * m
    return g * (a - m) * lax.rsqrt(v + eps) + b


def _deg_body(eiF, dp, part, ibuf, shared, accbuf, tbuf):
    c = lax.axis_index("c")
    s = lax.axis_index("s")
    z16 = jnp.zeros((16,), jnp.float32)
    ones = jnp.ones((16,), jnp.float32)

    def zloop(i, _):
        part[pl.ds(i * 16, 16)] = z16
        return 0
    lax.fori_loop(0, NPAD // 16, zloop, 0)

    def chunk(k, _):
        pltpu.sync_copy(eiF.at[c, s, k], ibuf)

        def inner(i, _):
            idx = ibuf[pl.ds(i * 16, 16)]
            plsc.addupdate_scatter(part, [idx], ones)
            return 0
        lax.fori_loop(0, CH // 16, inner, 0)
        return 0
    lax.fori_loop(0, NCHUNK, chunk, 0)

    pltpu.sync_copy(part, shared.at[s])
    plsc.subcore_barrier()

    base = s * TILE_N
    pltpu.sync_copy(shared.at[0, pl.ds(base, TILE_N)], accbuf)
    for t in range(1, NT):
        pltpu.sync_copy(shared.at[t, pl.ds(base, TILE_N)], tbuf)

        def addloop(i, _):
            sl = pl.ds(i * 16, 16)
            accbuf[sl] = accbuf[sl] + tbuf[sl]
            return 0
        lax.fori_loop(0, TILE_N // 16, addloop, 0)
    pltpu.sync_copy(accbuf, dp.at[c, pl.ds(base, TILE_N)])


_deg = pl.kernel(
    _deg_body,
    out_type=jax.ShapeDtypeStruct((NC, NPAD), jnp.float32),
    mesh=_mesh,
    scratch_types=[
        pltpu.VMEM((NPAD,), jnp.float32),
        pltpu.VMEM((CH,), jnp.int32),
        pltpu.VMEM_SHARED((NT, NPAD), jnp.float32),
        pltpu.VMEM((TILE_N,), jnp.float32),
        pltpu.VMEM((TILE_N,), jnp.float32),
    ],
)


def _prop32_body(eiR, gA, gB, outA, outB, sidx, didx, rows, acc, gsem, ssem):
    c = lax.axis_index("c")
    s = lax.axis_index("s")
    z16 = jnp.zeros((16,), jnp.float32)

    def zr(i, _):
        rows[i, :] = z16
        return 0
    lax.fori_loop(0, CH, zr, 0)

    base = s * TILE_N
    for i in range(4):
        pltpu.sync_copy(rows.at[pl.ds(0, 1568)],
                        acc.at[pl.ds(base + i * 1568, 1568)])
    plsc.subcore_barrier()

    for d in range(2):
        def chunk(k, _, d=d):
            pltpu.sync_copy(eiR.at[d, s, k], sidx)
            pltpu.sync_copy(eiR.at[1 - d, s, k], didx)

            @pl.when(c == 0)
            def _():
                descs = [
                    pltpu.async_copy(gA.at[sidx.at[j]],
                                     rows.at[pl.ds(j * COLS, COLS)], gsem)
                    for j in range(ROWS)
                ]
                for dsc in descs:
                    dsc.wait()

            @pl.when(c == 1)
            def _():
                descs = [
                    pltpu.async_copy(gB.at[sidx.at[j]],
                                     rows.at[pl.ds(j * COLS, COLS)], gsem)
                    for j in range(ROWS)
                ]
                for dsc in descs:
                    dsc.wait()

            descs = [
                pltpu.async_copy(rows.at[pl.ds(j * COLS, COLS)],
                                 acc.at[didx.at[j]], ssem, add=True)
                for j in range(ROWS)
            ]
            for dsc in descs:
                dsc.wait()
            return 0
        lax.fori_loop(0, NCHUNK, chunk, 0)

    plsc.subcore_barrier()

    @pl.when(c == 0)
    def _():
        for i in range(4):
            pltpu.sync_copy(acc.at[pl.ds(base + i * 1568, 1568)],
                            outA.at[pl.ds(base + i * 1568, 1568)])

    @pl.when(c == 1)
    def _():
        for i in range(4):
            pltpu.sync_copy(acc.at[pl.ds(base + i * 1568, 1568)],
                            outB.at[pl.ds(base + i * 1568, 1568)])


_prop32 = pl.kernel(
    _prop32_body,
    out_type=(jax.ShapeDtypeStruct((NPAD, 16), jnp.float32),
              jax.ShapeDtypeStruct((NPAD, 16), jnp.float32)),
    mesh=_mesh,
    scratch_types=[
        pltpu.VMEM((ROWS, COLS), jnp.int32),
        pltpu.VMEM((ROWS, COLS), jnp.int32),
        pltpu.VMEM((CH, 16), jnp.float32),
        pltpu.VMEM_SHARED((NPAD, 16), jnp.float32),
        pltpu.SemaphoreType.DMA,
        pltpu.SemaphoreType.DMA,
    ],
)


def _prop1_body(eiR, eiF, g2, tp, part, sidx, dbuf, vbuf, shared, accbuf,
                tbuf, sem):
    c = lax.axis_index("c")
    s = lax.axis_index("s")
    z16 = jnp.zeros((16,), jnp.float32)

    def zloop(i, _):
        part[pl.ds(i * 16, 16)] = z16
        return 0
    lax.fori_loop(0, NPAD // 16, zloop, 0)

    def chunk(k, _):
        pltpu.sync_copy(eiR.at[c, s, k], sidx)
        pltpu.sync_copy(eiF.at[1 - c, s, k], dbuf)
        descs = [
            pltpu.async_copy(g2.at[sidx.at[j]],
                             vbuf.at[pl.ds(j * COLS, COLS)], sem)
            for j in range(ROWS)
        ]
        for dsc in descs:
            dsc.wait()

        def inner(i, _):
            sl = pl.ds(i * 16, 16)
            plsc.addupdate_scatter(part, [dbuf[sl]], vbuf[sl])
            return 0
        lax.fori_loop(0, CH // 16, inner, 0)
        return 0
    lax.fori_loop(0, NCHUNK, chunk, 0)

    pltpu.sync_copy(part, shared.at[s])
    plsc.subcore_barrier()

    base = s * TILE_N
    pltpu.sync_copy(shared.at[0, pl.ds(base, TILE_N)], accbuf)
    for t in range(1, NT):
        pltpu.sync_copy(shared.at[t, pl.ds(base, TILE_N)], tbuf)

        def addloop(i, _):
            sl = pl.ds(i * 16, 16)
            accbuf[sl] = accbuf[sl] + tbuf[sl]
            return 0
        lax.fori_loop(0, TILE_N // 16, addloop, 0)
    pltpu.sync_copy(accbuf, tp.at[c, pl.ds(base, TILE_N)])


_prop1 = pl.kernel(
    _prop1_body,
    out_type=jax.ShapeDtypeStruct((NC, NPAD), jnp.float32),
    mesh=_mesh,
    scratch_types=[
        pltpu.VMEM((NPAD,), jnp.float32),
        pltpu.VMEM((ROWS, COLS), jnp.int32),
        pltpu.VMEM((CH,), jnp.int32),
        pltpu.VMEM((CH,), jnp.float32),
        pltpu.VMEM_SHARED((NT, NPAD), jnp.float32),
        pltpu.VMEM((TILE_N,), jnp.float32),
        pltpu.VMEM((TILE_N,), jnp.float32),
        pltpu.SemaphoreType.DMA,
    ],
)


def _dense_a_body(x_r, w1_r, b1_r, w2_r, b2_r, w3_r, b3_r,
                  g1_r, gb1_r, g2_r, gb2_r, g3_r, gb3_r,
                  wc1_r, d0_r, d1_r, gA_r, gB_r, dinv_r):
    x = x_r[...]
    a1 = jnp.dot(x, w1_r[...], preferred_element_type=jnp.float32) + b1_r[...]
    h1 = _lrelu(_bn(a1, g1_r[...], gb1_r[...]))
    a2 = jnp.dot(h1, w2_r[...], preferred_element_type=jnp.float32) + b2_r[...]
    h2 = _lrelu(_bn(a2, g2_r[...], gb2_r[...]))
    a3 = jnp.dot(h2, w3_r[...], preferred_element_type=jnp.float32) + b3_r[...]
    h3 = _lrelu(_bn(a3, g3_r[...], gb3_r[...]))
    h1p = jnp.dot(h3, wc1_r[...], preferred_element_type=jnp.float32)
    deg = d0_r[...][:N] + d1_r[...][:N] + 1.0
    dinv = lax.rsqrt(deg)
    g = dinv * h1p
    gA_r[...] = g[:, :16]
    gB_r[...] = g[:, 16:]
    dinv_r[...] = dinv


def _dense_b_body(sA_r, sB_r, gA_r, gB_r, dinv_r, bc1_r, wc2_r, g2_r):
    dinv = dinv_r[...]
    zA = _lrelu(dinv * (sA_r[...][:N] + gA_r[...]) + bc1_r[...][:, :16])
    zB = _lrelu(dinv * (sB_r[...][:N] + gB_r[...]) + bc1_r[...][:, 16:])
    y2 = (jnp.dot(zA, wc2_r[...][:16], preferred_element_type=jnp.float32)
          + jnp.dot(zB, wc2_r[...][16:], preferred_element_type=jnp.float32))
    g2_r[...] = dinv * y2


def _dense_c_body(t0_r, t1_r, g2_r, dinv_r, bc2_r, out_r):
    t = t0_r[...][:N] + t1_r[...][:N] + g2_r[...]
    out_r[...] = dinv_r[...] * t + bc2_r[...]


def kernel(x, edge_index, fc1_W, fc1_b, fc2_W, fc2_b, fc3_W, fc3_b,
           bn1_g, bn1_b, bn2_g, bn2_b, bn3_g, bn3_b,
           conv1_W, conv1_b, conv2_W, conv2_b):
    f32 = jnp.float32
    eiR = edge_index.reshape(2, NT, NCHUNK, ROWS, COLS)
    eiF = edge_index.reshape(2, NT, NCHUNK, CH)

    dp = _deg(eiF)
    d0 = dp[0].reshape(NPAD, 1)
    d1 = dp[1].reshape(NPAD, 1)

    gA, gB, dinv = pl.pallas_call(
        _dense_a_body,
        out_shape=(jax.ShapeDtypeStruct((N, 16), f32),
                   jax.ShapeDtypeStruct((N, 16), f32),
                   jax.ShapeDtypeStruct((N, 1), f32)),
    )(x, fc1_W.T, fc1_b, fc2_W.T, fc2_b, fc3_W.T, fc3_b,
      bn1_g, bn1_b, bn2_g, bn2_b, bn3_g, bn3_b,
      conv1_W.T, d0, d1)

    sA, sB = _prop32(eiR, gA, gB)

    g2 = pl.pallas_call(
        _dense_b_body,
        out_shape=jax.ShapeDtypeStruct((N, 1), f32),
    )(sA, sB, gA, gB, dinv, conv1_b.reshape(1, 32), conv2_W.T)

    tp = _prop1(eiR, eiF, g2.reshape(N))
    t0 = tp[0].reshape(NPAD, 1)
    t1 = tp[1].reshape(NPAD, 1)

    out = pl.pallas_call(
        _dense_c_body,
        out_shape=jax.ShapeDtypeStruct((N, 1), f32),
    )(t0, t1, g2, dinv, conv2_b.reshape(1, 1))
    return out


# speedup vs baseline: 44.7363x; 44.7363x over previous
import jax
import jax.numpy as jnp
from jax import lax
from jax.experimental import pallas as pl
from jax.experimental.pallas import tpu as pltpu
from jax.experimental.pallas import tpu_sc as plsc

N = 100000
E = 1600000
NT = 16
NC = 2
CH = 2000
ROWS = 25
COLS = 80
NCHUNK = E // NT // CH
NPAD = 100352
TILE_N = NPAD // NT
F = 8
BR = 6272
NB = NPAD // BR
BRB = 2048
NBB = NPAD // BRB

_mesh = plsc.VectorSubcoreMesh(core_axis_name="c", subcore_axis_name="s")
_sc_params = pltpu.CompilerParams(needs_layout_passes=False,
                                  use_tc_tiling_on_sc=False)


def _lrelu(h):
    return jnp.where(h >= 0, h, 0.1 * h)


def _reduce_tiles(src_vmem, part_hbm, out_hbm, c, s, accbuf, tbuf):
    base = s * TILE_N
    pltpu.sync_copy(src_vmem, part_hbm.at[c, s])
    plsc.subcore_barrier()

    def addloop(i, _):
        sl = pl.ds(i * 16, 16)
        accbuf[sl] = accbuf[sl] + tbuf[sl]
        return 0

    pltpu.sync_copy(part_hbm.at[c, 0, pl.ds(base, TILE_N)], accbuf)
    for t in range(1, NT):
        pltpu.sync_copy(part_hbm.at[c, t, pl.ds(base, TILE_N)], tbuf)
        lax.fori_loop(0, TILE_N // 16, addloop, 0)
    pltpu.sync_copy(accbuf, out_hbm.at[c, pl.ds(base, TILE_N)])


def _deg_body(eiR, dp, dpart, part, ibuf, accbuf, tbuf):
    c = lax.axis_index("c")
    s = lax.axis_index("s")
    z16 = jnp.zeros((16,), jnp.float32)
    ones = jnp.ones((16,), jnp.float32)

    def zloop(i, _):
        part[pl.ds(i * 16, 16)] = z16
        return 0
    lax.fori_loop(0, NPAD // 16, zloop, 0)

    def chunk(k, _):
        pltpu.sync_copy(eiR.at[c, s, k], ibuf)

        def jloop(j, _):
            def tloop(t, _):
                idx = ibuf[j, pl.ds(t * 16, 16)]
                plsc.addupdate_scatter(part, [idx], ones)
                return 0
            lax.fori_loop(0, COLS // 16, tloop, 0)
            return 0
        lax.fori_loop(0, ROWS, jloop, 0)
        return 0
    lax.fori_loop(0, NCHUNK, chunk, 0)

    _reduce_tiles(part, dpart, dp, c, s, accbuf, tbuf)


_deg = pl.kernel(
    _deg_body,
    out_type=(jax.ShapeDtypeStruct((NC, NPAD), jnp.float32),
              jax.ShapeDtypeStruct((NC, NT, NPAD), jnp.float32)),
    mesh=_mesh,
    compiler_params=_sc_params,
    scratch_types=[
        pltpu.VMEM((NPAD,), jnp.float32),
        pltpu.VMEM((ROWS, COLS), jnp.int32),
        pltpu.VMEM((TILE_N,), jnp.float32),
        pltpu.VMEM((TILE_N,), jnp.float32),
    ],
)


def _prop8_body(eiR, gLo, gHi, zrows, outLo, outHi, sidx, didx, rows, acc,
                gsem, ssem):
    c = lax.axis_index("c")
    s = lax.axis_index("s")

    base = s * TILE_N
    pltpu.sync_copy(zrows, acc.at[pl.ds(base, TILE_N)])
    plsc.subcore_barrier()

    for d in range(2):
        def chunk(k, _, d=d):
            pltpu.sync_copy(eiR.at[d, s, k], sidx)
            pltpu.sync_copy(eiR.at[1 - d, s, k], didx)

            @pl.when(c == 0)
            def _():
                descs = [
                    pltpu.async_copy(gLo.at[sidx.at[j]],
                                     rows.at[pl.ds(j * COLS, COLS)], gsem)
                    for j in range(ROWS)
                ]
                for dsc in descs:
                    dsc.wait()

            @pl.when(c == 1)
            def _():
                descs = [
                    pltpu.async_copy(gHi.at[sidx.at[j]],
                                     rows.at[pl.ds(j * COLS, COLS)], gsem)
                    for j in range(ROWS)
                ]
                for dsc in descs:
                    dsc.wait()

            descs = [
                pltpu.async_copy(rows.at[pl.ds(j * COLS, COLS)],
                                 acc.at[didx.at[j]], ssem, add=True)
                for j in range(ROWS)
            ]
            for dsc in descs:
                dsc.wait()
            return 0
        lax.fori_loop(0, NCHUNK, chunk, 0)

    plsc.subcore_barrier()

    @pl.when(c == 0)
    def _():
        for i in range(4):
            pltpu.sync_copy(acc.at[pl.ds(base + i * 1568, 1568)],
                            outLo.at[pl.ds(base + i * 1568, 1568)])

    @pl.when(c == 1)
    def _():
        for i in range(4):
            pltpu.sync_copy(acc.at[pl.ds(base + i * 1568, 1568)],
                            outHi.at[pl.ds(base + i * 1568, 1568)])


_prop8 = pl.kernel(
    _prop8_body,
    out_type=(jax.ShapeDtypeStruct((NPAD, F), jnp.float32),
              jax.ShapeDtypeStruct((NPAD, F), jnp.float32)),
    mesh=_mesh,
    compiler_params=_sc_params,
    scratch_types=[
        pltpu.VMEM((ROWS, COLS), jnp.int32),
        pltpu.VMEM((ROWS, COLS), jnp.int32),
        pltpu.VMEM((CH, F), jnp.float32),
        pltpu.VMEM_SHARED((NPAD, F), jnp.float32),
        pltpu.SemaphoreType.DMA,
        pltpu.SemaphoreType.DMA,
    ],
)


def _prop1_body(eiR, g2, tp, tpart, part, sidx, dbuf, vbuf, accbuf,
                tbuf, sem):
    c = lax.axis_index("c")
    s = lax.axis_index("s")
    z16 = jnp.zeros((16,), jnp.float32)

    def zloop(i, _):
        part[pl.ds(i * 16, 16)] = z16
        return 0
    lax.fori_loop(0, NPAD // 16, zloop, 0)

    def chunk(k, _):
        pltpu.sync_copy(eiR.at[c, s, k], sidx)
        pltpu.sync_copy(eiR.at[1 - c, s, k], dbuf)
        descs = [
            pltpu.async_copy(g2.at[sidx.at[j]],
                             vbuf.at[pl.ds(j * COLS, COLS)], sem)
            for j in range(ROWS)
        ]
        for dsc in descs:
            dsc.wait()

        def jloop(j, _):
            def tloop(t, _):
                vals = vbuf[pl.ds(j * COLS + t * 16, 16)]
                idx = dbuf[j, pl.ds(t * 16, 16)]
                plsc.addupdate_scatter(part, [idx], vals)
                return 0
            lax.fori_loop(0, COLS // 16, tloop, 0)
            return 0
        lax.fori_loop(0, ROWS, jloop, 0)
        return 0
    lax.fori_loop(0, NCHUNK, chunk, 0)

    _reduce_tiles(part, tpart, tp, c, s, accbuf, tbuf)


_prop1 = pl.kernel(
    _prop1_body,
    out_type=(jax.ShapeDtypeStruct((NC, NPAD), jnp.float32),
              jax.ShapeDtypeStruct((NC, NT, NPAD), jnp.float32)),
    mesh=_mesh,
    compiler_params=_sc_params,
    scratch_types=[
        pltpu.VMEM((NPAD,), jnp.float32),
        pltpu.VMEM((ROWS, COLS), jnp.int32),
        pltpu.VMEM((ROWS, COLS), jnp.int32),
        pltpu.VMEM((CH,), jnp.float32),
        pltpu.VMEM((TILE_N,), jnp.float32),
        pltpu.VMEM((TILE_N,), jnp.float32),
        pltpu.SemaphoreType.DMA,
    ],
)


def _dense_a_body(xT_r, w1_r, b1_r, w2_r, b2_r, w3_r, b3_r,
                  g1_r, gb1_r, g2_r, gb2_r, g3_r, gb3_r,
                  wc1_r, dp_r,
                  g0_r, g1o_r, g2o_r, g3o_r, dinv_r,
                  s1_r, s2_r, s3_r, acc_r, st_r):
    p = pl.program_id(0)
    b = pl.program_id(1)
    eps = 1e-5
    glob = b * BR + lax.broadcasted_iota(jnp.int32, (1, BR), 1)
    msk = glob < N

    @pl.when((p == 0) & (b == 0))
    def _():
        acc_r[...] = jnp.zeros((64, 2), jnp.float32)

    @pl.when(p == 0)
    def _():
        a1 = jnp.dot(w1_r[...], xT_r[...],
                     preferred_element_type=jnp.float32) + b1_r[...]
        s1_r[b] = a1
        a1m = jnp.where(msk, a1, 0.0)
        acc_r[0:8, 0:1] += jnp.sum(a1m, axis=1, keepdims=True)
        acc_r[0:8, 1:2] += jnp.sum(a1m * a1m, axis=1, keepdims=True)

    @pl.when((p == 1) & (b == 0))
    def _():
        m = acc_r[0:8, 0:1] / N
        v = acc_r[0:8, 1:2] / N - m * m
        st_r[0:8, 0:1] = m
        st_r[0:8, 1:2] = lax.rsqrt(v + eps)

    @pl.when(p == 1)
    def _():
        h1 = _lrelu(g1_r[...] * (s1_r[b] - st_r[0:8, 0:1])
                    * st_r[0:8, 1:2] + gb1_r[...])
        a2 = jnp.dot(w2_r[...], h1,
                     preferred_element_type=jnp.float32) + b2_r[...]
        s2_r[b] = a2
        a2m = jnp.where(msk, a2, 0.0)
        acc_r[8:24, 0:1] += jnp.sum(a2m, axis=1, keepdims=True)
        acc_r[8:24, 1:2] += jnp.sum(a2m * a2m, axis=1, keepdims=True)

    @pl.when((p == 2) & (b == 0))
    def _():
        m = acc_r[8:24, 0:1] / N
        v = acc_r[8:24, 1:2] / N - m * m
        st_r[8:24, 0:1] = m
        st_r[8:24, 1:2] = lax.rsqrt(v + eps)

    @pl.when(p == 2)
    def _():
        h2 = _lrelu(g2_r[...] * (s2_r[b] - st_r[8:24, 0:1])
                    * st_r[8:24, 1:2] + gb2_r[...])
        a3 = jnp.dot(w3_r[...], h2,
                     preferred_element_type=jnp.float32) + b3_r[...]
        s3_r[b] = a3
        a3m = jnp.where(msk, a3, 0.0)
        acc_r[24:56, 0:1] += jnp.sum(a3m, axis=1, keepdims=True)
        acc_r[24:56, 1:2] += jnp.sum(a3m * a3m, axis=1, keepdims=True)

    @pl.when((p == 3) & (b == 0))
    def _():
        m = acc_r[24:56, 0:1] / N
        v = acc_r[24:56, 1:2] / N - m * m
        st_r[24:56, 0:1] = m
        st_r[24:56, 1:2] = lax.rsqrt(v + eps)

    @pl.when(p == 3)
    def _():
        h3 = _lrelu(g3_r[...] * (s3_r[b] - st_r[24:56, 0:1])
                    * st_r[24:56, 1:2] + gb3_r[...])
        h1pT = jnp.dot(wc1_r[...], h3, preferred_element_type=jnp.float32)
        deg = dp_r[0:1, :] + dp_r[1:2, :] + 1.0
        dinv = lax.rsqrt(deg)
        gT = dinv * h1pT
        gblk = jnp.transpose(gT)
        g0_r[...] = gblk[:, 0:8]
        g1o_r[...] = gblk[:, 8:16]
        g2o_r[...] = gblk[:, 16:24]
        g3o_r[...] = gblk[:, 24:32]
        dinv_r[...] = dinv


def _dense_b_body(s0_r, s1_r, s2_r, s3_r, g0_r, g1_r, g2_r, g3_r,
                  dinv_r, bc1_r, wc2_r, out_r):
    dinv = jnp.transpose(dinv_r[...])
    y2 = jnp.zeros((BRB, 1), jnp.float32)
    for i, (s_r, g_r) in enumerate(((s0_r, g0_r), (s1_r, g1_r),
                                    (s2_r, g2_r), (s3_r, g3_r))):
        z = _lrelu(dinv * (s_r[...] + g_r[...])
                   + bc1_r[...][:, i * 8:(i + 1) * 8])
        y2 = y2 + jnp.dot(z, wc2_r[...][i * 8:(i + 1) * 8],
                          preferred_element_type=jnp.float32)
    out_r[...] = jnp.transpose(dinv * y2)


def _dense_c_body(tp_r, g2_r, dinv_r, bc2_r, out_r):
    t = tp_r[0:1, :] + tp_r[1:2, :] + g2_r[...]
    out_r[...] = jnp.transpose(dinv_r[...] * t + bc2_r[...])


def kernel(x, edge_index, fc1_W, fc1_b, fc2_W, fc2_b, fc3_W, fc3_b,
           bn1_g, bn1_b, bn2_g, bn2_b, bn3_g, bn3_b,
           conv1_W, conv1_b, conv2_W, conv2_b):
    f32 = jnp.float32
    eiR = edge_index.reshape(2, NT, NCHUNK, ROWS, COLS)

    dp, _dpart = _deg(eiR)

    col = lambda v: v.reshape(-1, 1)
    full = lambda shape: pl.BlockSpec(shape, lambda p, b: (0,) * len(shape))
    g_spec = pl.BlockSpec((BR, F), lambda p, b: (b, 0))
    lane_spec = pl.BlockSpec((2, BR), lambda p, b: (0, b))

    xT = jnp.pad(x.T, ((0, 0), (0, NPAD - N)))
    g0, g1, g2s, g3s, dinv = pl.pallas_call(
        _dense_a_body,
        grid=(4, NB),
        in_specs=[
            lane_spec,
            full((8, 2)), full((8, 1)),
            full((16, 8)), full((16, 1)),
            full((32, 16)), full((32, 1)),
            full((8, 1)), full((8, 1)),
            full((16, 1)), full((16, 1)),
            full((32, 1)), full((32, 1)),
            full((32, 32)),
            lane_spec,
        ],
        out_specs=[g_spec, g_spec, g_spec, g_spec,
                   pl.BlockSpec((1, BR), lambda p, b: (0, b))],
        out_shape=(jax.ShapeDtypeStruct((NPAD, F), f32),
                   jax.ShapeDtypeStruct((NPAD, F), f32),
                   jax.ShapeDtypeStruct((NPAD, F), f32),
                   jax.ShapeDtypeStruct((NPAD, F), f32),
                   jax.ShapeDtypeStruct((1, NPAD), f32)),
        scratch_shapes=[
            pltpu.VMEM((NB, 8, BR), f32),
            pltpu.VMEM((NB, 16, BR), f32),
            pltpu.VMEM((NB, 32, BR), f32),
            pltpu.VMEM((64, 2), f32),
            pltpu.VMEM((64, 2), f32),
        ],
    )(xT, fc1_W, col(fc1_b), fc2_W, col(fc2_b), fc3_W, col(fc3_b),
      col(bn1_g), col(bn1_b), col(bn2_g), col(bn2_b), col(bn3_g),
      col(bn3_b), conv1_W, dp)

    zrows = jnp.zeros((TILE_N, F), f32)
    s0, s1 = _prop8(eiR, g0, g1, zrows)
    s2, s3 = _prop8(eiR, g2s, g3s, zrows)

    gb_spec = pl.BlockSpec((BRB, F), lambda b: (b, 0))
    row1_spec = pl.BlockSpec((1, BRB), lambda b: (0, b))
    fullb = lambda shape: pl.BlockSpec(shape, lambda b: (0,) * len(shape))

    g2 = pl.pallas_call(
        _dense_b_body,
        grid=(NBB,),
        in_specs=[gb_spec, gb_spec, gb_spec, gb_spec,
                  gb_spec, gb_spec, gb_spec, gb_spec,
                  row1_spec, fullb((1, 32)), fullb((32, 1))],
        out_specs=row1_spec,
        out_shape=jax.ShapeDtypeStruct((1, NPAD), f32),
    )(s0, s1, s2, s3, g0, g1, g2s, g3s, dinv,
      conv1_b.reshape(1, 32), conv2_W.T)

    tp, _tpart = _prop1(eiR, g2.reshape(NPAD))

    out = pl.pallas_call(
        _dense_c_body,
        grid=(NBB,),
        in_specs=[pl.BlockSpec((2, BRB), lambda b: (0, b)),
                  row1_spec, row1_spec, fullb((1, 1))],
        out_specs=pl.BlockSpec((BRB, 1), lambda b: (b, 0)),
        out_shape=jax.ShapeDtypeStruct((NPAD, 1), f32),
    )(tp, g2, dinv, conv2_b.reshape(1, 1))
    return out[:N]


# speedup vs baseline: 50.2380x; 1.1230x over previous
import jax
import jax.numpy as jnp
from jax import lax
from jax.experimental import pallas as pl
from jax.experimental.pallas import tpu as pltpu
from jax.experimental.pallas import tpu_sc as plsc

N = 100000
E = 1600000
NT = 16
NC = 2
CH = 800
ROWS = 10
COLS = 80
NCHUNK = E // NT // CH
NPAD = 100352
TILE_N = NPAD // NT
F = 16
BR = 6272
NB = NPAD // BR
BRB = 2048
NBB = NPAD // BRB

_mesh = plsc.VectorSubcoreMesh(core_axis_name="c", subcore_axis_name="s")
_sc_params = pltpu.CompilerParams(needs_layout_passes=False,
                                  use_tc_tiling_on_sc=False)


def _lrelu(h):
    return jnp.where(h >= 0, h, 0.1 * h)


def _reduce_tiles(src_vmem, part_hbm, out_hbm, c, s, accbuf, tbuf):
    base = s * TILE_N
    pltpu.sync_copy(src_vmem, part_hbm.at[c, s])
    plsc.subcore_barrier()

    def addloop(i, _):
        sl = pl.ds(i * 16, 16)
        accbuf[sl] = accbuf[sl] + tbuf[sl]
        return 0

    pltpu.sync_copy(part_hbm.at[c, 0, pl.ds(base, TILE_N)], accbuf)
    for t in range(1, NT):
        pltpu.sync_copy(part_hbm.at[c, t, pl.ds(base, TILE_N)], tbuf)
        lax.fori_loop(0, TILE_N // 16, addloop, 0)
    pltpu.sync_copy(accbuf, out_hbm.at[c, pl.ds(base, TILE_N)])


def _deg_body(eiR, dp, dpart, part, ibuf, accbuf, tbuf):
    c = lax.axis_index("c")
    s = lax.axis_index("s")
    z16 = jnp.zeros((16,), jnp.float32)
    ones = jnp.ones((16,), jnp.float32)

    def zloop(i, _):
        part[pl.ds(i * 16, 16)] = z16
        return 0
    lax.fori_loop(0, NPAD // 16, zloop, 0)

    def chunk(k, _):
        pltpu.sync_copy(eiR.at[c, s, k], ibuf)

        def jloop(j, _):
            def tloop(t, _):
                idx = ibuf[j, pl.ds(t * 16, 16)]
                plsc.addupdate_scatter(part, [idx], ones)
                return 0
            lax.fori_loop(0, COLS // 16, tloop, 0)
            return 0
        lax.fori_loop(0, ROWS, jloop, 0)
        return 0
    lax.fori_loop(0, NCHUNK, chunk, 0)

    _reduce_tiles(part, dpart, dp, c, s, accbuf, tbuf)


_deg = pl.kernel(
    _deg_body,
    out_type=(jax.ShapeDtypeStruct((NC, NPAD), jnp.float32),
              jax.ShapeDtypeStruct((NC, NT, NPAD), jnp.float32)),
    mesh=_mesh,
    compiler_params=_sc_params,
    scratch_types=[
        pltpu.VMEM((NPAD,), jnp.float32),
        pltpu.VMEM((ROWS, COLS), jnp.int32),
        pltpu.VMEM((TILE_N,), jnp.float32),
        pltpu.VMEM((TILE_N,), jnp.float32),
    ],
)


def _prop16_body(eiR, gLo, gHi, zrows, outLo, outHi, sidx, didx, rows, acc,
                 gsem, ssem):
    c = lax.axis_index("c")
    s = lax.axis_index("s")

    base = s * TILE_N
    pltpu.sync_copy(zrows, acc.at[pl.ds(base, TILE_N)])
    plsc.subcore_barrier()

    for d in range(2):
        def chunk(k, _, d=d):
            pltpu.sync_copy(eiR.at[d, s, k], sidx)
            pltpu.sync_copy(eiR.at[1 - d, s, k], didx)

            @pl.when(c == 0)
            def _():
                descs = [
                    pltpu.async_copy(gLo.at[sidx.at[j]],
                                     rows.at[pl.ds(j * COLS, COLS)], gsem)
                    for j in range(ROWS)
                ]
                for dsc in descs:
                    dsc.wait()

            @pl.when(c == 1)
            def _():
                descs = [
                    pltpu.async_copy(gHi.at[sidx.at[j]],
                                     rows.at[pl.ds(j * COLS, COLS)], gsem)
                    for j in range(ROWS)
                ]
                for dsc in descs:
                    dsc.wait()

            descs = [
                pltpu.async_copy(rows.at[pl.ds(j * COLS, COLS)],
                                 acc.at[didx.at[j]], ssem, add=True)
                for j in range(ROWS)
            ]
            for dsc in descs:
                dsc.wait()
            return 0
        lax.fori_loop(0, NCHUNK, chunk, 0)

    plsc.subcore_barrier()

    @pl.when(c == 0)
    def _():
        for i in range(4):
            pltpu.sync_copy(acc.at[pl.ds(base + i * 1568, 1568)],
                            outLo.at[pl.ds(base + i * 1568, 1568)])

    @pl.when(c == 1)
    def _():
        for i in range(4):
            pltpu.sync_copy(acc.at[pl.ds(base + i * 1568, 1568)],
                            outHi.at[pl.ds(base + i * 1568, 1568)])


_prop16 = pl.kernel(
    _prop16_body,
    out_type=(jax.ShapeDtypeStruct((NPAD, F), jnp.float32),
              jax.ShapeDtypeStruct((NPAD, F), jnp.float32)),
    mesh=_mesh,
    compiler_params=_sc_params,
    scratch_types=[
        pltpu.VMEM((ROWS, COLS), jnp.int32),
        pltpu.VMEM((ROWS, COLS), jnp.int32),
        pltpu.VMEM((CH, F), jnp.float32),
        pltpu.VMEM_SHARED((NPAD, F), jnp.float32),
        pltpu.SemaphoreType.DMA,
        pltpu.SemaphoreType.DMA,
    ],
)


def _prop1_body(eiR, g2, tp, tpart, part, sidx, dbuf, vbuf, accbuf,
                tbuf, sem):
    c = lax.axis_index("c")
    s = lax.axis_index("s")
    z16 = jnp.zeros((16,), jnp.float32)

    def zloop(i, _):
        part[pl.ds(i * 16, 16)] = z16
        return 0
    lax.fori_loop(0, NPAD // 16, zloop, 0)

    def chunk(k, _):
        pltpu.sync_copy(eiR.at[c, s, k], sidx)
        pltpu.sync_copy(eiR.at[1 - c, s, k], dbuf)
        descs = [
            pltpu.async_copy(g2.at[sidx.at[j]],
                             vbuf.at[pl.ds(j * COLS, COLS)], sem)
            for j in range(ROWS)
        ]
        for dsc in descs:
            dsc.wait()

        def jloop(j, _):
            def tloop(t, _):
                vals = vbuf[pl.ds(j * COLS + t * 16, 16)]
                idx = dbuf[j, pl.ds(t * 16, 16)]
                plsc.addupdate_scatter(part, [idx], vals)
                return 0
            lax.fori_loop(0, COLS // 16, tloop, 0)
            return 0
        lax.fori_loop(0, ROWS, jloop, 0)
        return 0
    lax.fori_loop(0, NCHUNK, chunk, 0)

    _reduce_tiles(part, tpart, tp, c, s, accbuf, tbuf)


_prop1 = pl.kernel(
    _prop1_body,
    out_type=(jax.ShapeDtypeStruct((NC, NPAD), jnp.float32),
              jax.ShapeDtypeStruct((NC, NT, NPAD), jnp.float32)),
    mesh=_mesh,
    compiler_params=_sc_params,
    scratch_types=[
        pltpu.VMEM((NPAD,), jnp.float32),
        pltpu.VMEM((ROWS, COLS), jnp.int32),
        pltpu.VMEM((ROWS, COLS), jnp.int32),
        pltpu.VMEM((CH,), jnp.float32),
        pltpu.VMEM((TILE_N,), jnp.float32),
        pltpu.VMEM((TILE_N,), jnp.float32),
        pltpu.SemaphoreType.DMA,
    ],
)


def _dense_a_body(xT_r, w1_r, b1_r, w2_r, b2_r, w3_r, b3_r,
                  g1_r, gb1_r, g2_r, gb2_r, g3_r, gb3_r,
                  wc1_r, dp_r,
                  g0_r, g1o_r, dinv_r,
                  s1_r, s2_r, s3_r, acc_r, st_r):
    p = pl.program_id(0)
    b = pl.program_id(1)
    eps = 1e-5
    glob = b * BR + lax.broadcasted_iota(jnp.int32, (1, BR), 1)
    msk = glob < N

    @pl.when((p == 0) & (b == 0))
    def _():
        acc_r[...] = jnp.zeros((64, 2), jnp.float32)

    @pl.when(p == 0)
    def _():
        a1 = jnp.dot(w1_r[...], xT_r[...],
                     preferred_element_type=jnp.float32) + b1_r[...]
        s1_r[b] = a1
        a1m = jnp.where(msk, a1, 0.0)
        acc_r[0:8, 0:1] += jnp.sum(a1m, axis=1, keepdims=True)
        acc_r[0:8, 1:2] += jnp.sum(a1m * a1m, axis=1, keepdims=True)

    @pl.when((p == 1) & (b == 0))
    def _():
        m = acc_r[0:8, 0:1] / N
        v = acc_r[0:8, 1:2] / N - m * m
        st_r[0:8, 0:1] = m
        st_r[0:8, 1:2] = lax.rsqrt(v + eps)

    @pl.when(p == 1)
    def _():
        h1 = _lrelu(g1_r[...] * (s1_r[b] - st_r[0:8, 0:1])
                    * st_r[0:8, 1:2] + gb1_r[...])
        a2 = jnp.dot(w2_r[...], h1,
                     preferred_element_type=jnp.float32) + b2_r[...]
        s2_r[b] = a2
        a2m = jnp.where(msk, a2, 0.0)
        acc_r[8:24, 0:1] += jnp.sum(a2m, axis=1, keepdims=True)
        acc_r[8:24, 1:2] += jnp.sum(a2m * a2m, axis=1, keepdims=True)

    @pl.when((p == 2) & (b == 0))
    def _():
        m = acc_r[8:24, 0:1] / N
        v = acc_r[8:24, 1:2] / N - m * m
        st_r[8:24, 0:1] = m
        st_r[8:24, 1:2] = lax.rsqrt(v + eps)

    @pl.when(p == 2)
    def _():
        h2 = _lrelu(g2_r[...] * (s2_r[b] - st_r[8:24, 0:1])
                    * st_r[8:24, 1:2] + gb2_r[...])
        a3 = jnp.dot(w3_r[...], h2,
                     preferred_element_type=jnp.float32) + b3_r[...]
        s3_r[b] = a3
        a3m = jnp.where(msk, a3, 0.0)
        acc_r[24:56, 0:1] += jnp.sum(a3m, axis=1, keepdims=True)
        acc_r[24:56, 1:2] += jnp.sum(a3m * a3m, axis=1, keepdims=True)

    @pl.when((p == 3) & (b == 0))
    def _():
        m = acc_r[24:56, 0:1] / N
        v = acc_r[24:56, 1:2] / N - m * m
        st_r[24:56, 0:1] = m
        st_r[24:56, 1:2] = lax.rsqrt(v + eps)

    @pl.when(p == 3)
    def _():
        h3 = _lrelu(g3_r[...] * (s3_r[b] - st_r[24:56, 0:1])
                    * st_r[24:56, 1:2] + gb3_r[...])
        h1pT = jnp.dot(wc1_r[...], h3, preferred_element_type=jnp.float32)
        deg = dp_r[0:1, :] + dp_r[1:2, :] + 1.0
        dinv = lax.rsqrt(deg)
        gT = dinv * h1pT
        gblk = jnp.transpose(gT)
        g0_r[...] = gblk[:, 0:16]
        g1o_r[...] = gblk[:, 16:32]
        dinv_r[...] = dinv


def _dense_b_body(s0_r, s1_r, g0_r, g1_r,
                  dinv_r, bc1_r, wc2_r, out_r):
    dinv = jnp.transpose(dinv_r[...])
    y2 = jnp.zeros((BRB, 1), jnp.float32)
    for i, (s_r, g_r) in enumerate(((s0_r, g0_r), (s1_r, g1_r))):
        z = _lrelu(dinv * (s_r[...] + g_r[...])
                   + bc1_r[...][:, i * 16:(i + 1) * 16])
        y2 = y2 + jnp.dot(z, wc2_r[...][i * 16:(i + 1) * 16],
                          preferred_element_type=jnp.float32)
    out_r[...] = jnp.transpose(dinv * y2)


def _dense_c_body(tp_r, g2_r, dinv_r, bc2_r, out_r):
    t = tp_r[0:1, :] + tp_r[1:2, :] + g2_r[...]
    out_r[...] = jnp.transpose(dinv_r[...] * t + bc2_r[...])


def kernel(x, edge_index, fc1_W, fc1_b, fc2_W, fc2_b, fc3_W, fc3_b,
           bn1_g, bn1_b, bn2_g, bn2_b, bn3_g, bn3_b,
           conv1_W, conv1_b, conv2_W, conv2_b):
    f32 = jnp.float32
    eiR = edge_index.reshape(2, NT, NCHUNK, ROWS, COLS)

    dp, _dpart = _deg(eiR)

    col = lambda v: v.reshape(-1, 1)
    full = lambda shape: pl.BlockSpec(shape, lambda p, b: (0,) * len(shape))
    g_spec = pl.BlockSpec((BR, F), lambda p, b: (b, 0))
    lane_spec = pl.BlockSpec((2, BR), lambda p, b: (0, b))

    xT = jnp.pad(x.T, ((0, 0), (0, NPAD - N)))
    g0, g1, dinv = pl.pallas_call(
        _dense_a_body,
        grid=(4, NB),
        in_specs=[
            lane_spec,
            full((8, 2)), full((8, 1)),
            full((16, 8)), full((16, 1)),
            full((32, 16)), full((32, 1)),
            full((8, 1)), full((8, 1)),
            full((16, 1)), full((16, 1)),
            full((32, 1)), full((32, 1)),
            full((32, 32)),
            lane_spec,
        ],
        out_specs=[g_spec, g_spec,
                   pl.BlockSpec((1, BR), lambda p, b: (0, b))],
        out_shape=(jax.ShapeDtypeStruct((NPAD, F), f32),
                   jax.ShapeDtypeStruct((NPAD, F), f32),
                   jax.ShapeDtypeStruct((1, NPAD), f32)),
        scratch_shapes=[
            pltpu.VMEM((NB, 8, BR), f32),
            pltpu.VMEM((NB, 16, BR), f32),
            pltpu.VMEM((NB, 32, BR), f32),
            pltpu.VMEM((64, 2), f32),
            pltpu.VMEM((64, 2), f32),
        ],
    )(xT, fc1_W, col(fc1_b), fc2_W, col(fc2_b), fc3_W, col(fc3_b),
      col(bn1_g), col(bn1_b), col(bn2_g), col(bn2_b), col(bn3_g),
      col(bn3_b), conv1_W, dp)

    zrows = jnp.zeros((TILE_N, F), f32)
    s0, s1 = _prop16(eiR, g0, g1, zrows)

    gb_spec = pl.BlockSpec((BRB, F), lambda b: (b, 0))
    row1_spec = pl.BlockSpec((1, BRB), lambda b: (0, b))
    fullb = lambda shape: pl.BlockSpec(shape, lambda b: (0,) * len(shape))

    g2 = pl.pallas_call(
        _dense_b_body,
        grid=(NBB,),
        in_specs=[gb_spec, gb_spec, gb_spec, gb_spec,
                  row1_spec, fullb((1, 32)), fullb((32, 1))],
        out_specs=row1_spec,
        out_shape=jax.ShapeDtypeStruct((1, NPAD), f32),
    )(s0, s1, g0, g1, dinv,
      conv1_b.reshape(1, 32), conv2_W.T)

    tp, _tpart = _prop1(eiR, g2.reshape(NPAD))

    out = pl.pallas_call(
        _dense_c_body,
        grid=(NBB,),
        in_specs=[pl.BlockSpec((2, BRB), lambda b: (0, b)),
                  row1_spec, row1_spec, fullb((1, 1))],
        out_specs=pl.BlockSpec((BRB, 1), lambda b: (b, 0)),
        out_shape=jax.ShapeDtypeStruct((NPAD, 1), f32),
    )(tp, g2, dinv, conv2_b.reshape(1, 1))
    return out[:N]


# speedup vs baseline: 51.1010x; 1.0172x over previous
import jax
import jax.numpy as jnp
from jax import lax
from jax.experimental import pallas as pl
from jax.experimental.pallas import tpu as pltpu
from jax.experimental.pallas import tpu_sc as plsc

N = 100000
E = 1600000
NT = 16
NC = 2
CH = 2000
ROWS = 25
COLS = 80
NSLOT = 10
DEPTH = 2
NCHUNK = E // NT // CH
NPAD = 100352
TILE_N = NPAD // NT
F = 16
BR = 6272
NB = NPAD // BR
BRB = 2048
NBB = NPAD // BRB

_mesh = plsc.VectorSubcoreMesh(core_axis_name="c", subcore_axis_name="s")
_sc_params = pltpu.CompilerParams(needs_layout_passes=False,
                                  use_tc_tiling_on_sc=False)


def _lrelu(h):
    return jnp.where(h >= 0, h, 0.1 * h)


def _reduce_tiles(src_vmem, part_hbm, out_hbm, c, s, accbuf, tbuf):
    base = s * TILE_N
    pltpu.sync_copy(src_vmem, part_hbm.at[c, s])
    plsc.subcore_barrier()

    def addloop(i, _):
        for u in range(8):
            sl = pl.ds(i * 128 + u * 16, 16)
            accbuf[sl] = accbuf[sl] + tbuf[sl]
        return 0

    pltpu.sync_copy(part_hbm.at[c, 0, pl.ds(base, TILE_N)], accbuf)
    for t in range(1, NT):
        pltpu.sync_copy(part_hbm.at[c, t, pl.ds(base, TILE_N)], tbuf)
        lax.fori_loop(0, TILE_N // 128, addloop, 0)
    pltpu.sync_copy(accbuf, out_hbm.at[c, pl.ds(base, TILE_N)])


def _deg_body(eiR, zcol, dp, dpart, part, ibuf, accbuf, tbuf):
    c = lax.axis_index("c")
    s = lax.axis_index("s")
    ones = jnp.ones((16,), jnp.float32)
    pltpu.sync_copy(zcol, part)

    def chunk(k, _):
        pltpu.sync_copy(eiR.at[c, s, k], ibuf)

        def jloop(j, _):
            for t in range(COLS // 16):
                idx = ibuf[j, pl.ds(t * 16, 16)]
                plsc.addupdate_scatter(part, [idx], ones)
            return 0
        lax.fori_loop(0, ROWS, jloop, 0)
        return 0
    lax.fori_loop(0, NCHUNK, chunk, 0)

    _reduce_tiles(part, dpart, dp, c, s, accbuf, tbuf)


_deg = pl.kernel(
    _deg_body,
    out_type=(jax.ShapeDtypeStruct((NC, NPAD), jnp.float32),
              jax.ShapeDtypeStruct((NC, NT, NPAD), jnp.float32)),
    mesh=_mesh,
    compiler_params=_sc_params,
    scratch_types=[
        pltpu.VMEM((NPAD,), jnp.float32),
        pltpu.VMEM((ROWS, COLS), jnp.int32),
        pltpu.VMEM((TILE_N,), jnp.float32),
        pltpu.VMEM((TILE_N,), jnp.float32),
    ],
)


def _prop16_body(eiR, gLo, gHi, zrows, outLo, outHi, sidx3, didx3, rows,
                 acc, gsem, ssem):
    c = lax.axis_index("c")
    s = lax.axis_index("s")

    base = s * TILE_N
    pltpu.sync_copy(zrows, acc.at[pl.ds(base, TILE_N)])
    plsc.subcore_barrier()

    def run_pipeline(g, sidx, didx):
        gd = {}
        sd = {}

        def fire_scatter(jj):
            gd[jj].wait()
            sd[jj] = pltpu.async_copy(
                rows.at[pl.ds((jj % NSLOT) * COLS, COLS)],
                acc.at[didx.at[jj]], ssem, add=True)

        for j in range(ROWS):
            if j >= NSLOT:
                sd[j - NSLOT].wait()
            gd[j] = pltpu.async_copy(
                g.at[sidx.at[j]],
                rows.at[pl.ds((j % NSLOT) * COLS, COLS)], gsem)
            if j >= DEPTH:
                fire_scatter(j - DEPTH)
        for jj in range(ROWS - DEPTH, ROWS):
            fire_scatter(jj)

    def drain_tail(didx):
        for j in range(NSLOT):
            pltpu.make_async_copy(
                rows.at[pl.ds(j * COLS, COLS)],
                acc.at[didx.at[0]], ssem).wait()

    for d in range(2):
        def chunk(k, _, d=d):
            kb = lax.rem(k, 2)
            sidx = sidx3.at[kb]
            didx = didx3.at[kb]
            pltpu.sync_copy(eiR.at[d, s, k], sidx)
            pltpu.sync_copy(eiR.at[1 - d, s, k], didx)

            if d == 0:
                @pl.when(k > 0)
                def _():
                    drain_tail(didx)
            else:
                drain_tail(didx)

            @pl.when(c == 0)
            def _():
                run_pipeline(gLo, sidx, didx)

            @pl.when(c == 1)
            def _():
                run_pipeline(gHi, sidx, didx)
            return 0
        lax.fori_loop(0, NCHUNK, chunk, 0)

    drain_tail(didx3.at[0])
    plsc.subcore_barrier()

    @pl.when(c == 0)
    def _():
        for i in range(4):
            pltpu.sync_copy(acc.at[pl.ds(base + i * 1568, 1568)],
                            outLo.at[pl.ds(base + i * 1568, 1568)])

    @pl.when(c == 1)
    def _():
        for i in range(4):
            pltpu.sync_copy(acc.at[pl.ds(base + i * 1568, 1568)],
                            outHi.at[pl.ds(base + i * 1568, 1568)])


_prop16 = pl.kernel(
    _prop16_body,
    out_type=(jax.ShapeDtypeStruct((NPAD, F), jnp.float32),
              jax.ShapeDtypeStruct((NPAD, F), jnp.float32)),
    mesh=_mesh,
    compiler_params=_sc_params,
    scratch_types=[
        pltpu.VMEM((2, ROWS, COLS), jnp.int32),
        pltpu.VMEM((2, ROWS, COLS), jnp.int32),
        pltpu.VMEM((NSLOT * COLS, F), jnp.float32),
        pltpu.VMEM_SHARED((NPAD, F), jnp.float32),
        pltpu.SemaphoreType.DMA,
        pltpu.SemaphoreType.DMA,
    ],
)


def _prop1_body(eiR, g2, zcol, tp, tpart, part, sidx, dbuf, vbuf, accbuf,
                tbuf, sem):
    c = lax.axis_index("c")
    s = lax.axis_index("s")
    pltpu.sync_copy(zcol, part)

    def chunk(k, _):
        pltpu.sync_copy(eiR.at[c, s, k], sidx)
        pltpu.sync_copy(eiR.at[1 - c, s, k], dbuf)
        descs = [
            pltpu.async_copy(g2.at[sidx.at[j]],
                             vbuf.at[pl.ds(j * COLS, COLS)], sem)
            for j in range(ROWS)
        ]
        for dsc in descs:
            dsc.wait()

        def jloop(j, _):
            for t in range(COLS // 16):
                vals = vbuf[pl.ds(j * COLS + t * 16, 16)]
                idx = dbuf[j, pl.ds(t * 16, 16)]
                plsc.addupdate_scatter(part, [idx], vals)
            return 0
        lax.fori_loop(0, ROWS, jloop, 0)
        return 0
    lax.fori_loop(0, NCHUNK, chunk, 0)

    _reduce_tiles(part, tpart, tp, c, s, accbuf, tbuf)


_prop1 = pl.kernel(
    _prop1_body,
    out_type=(jax.ShapeDtypeStruct((NC, NPAD), jnp.float32),
              jax.ShapeDtypeStruct((NC, NT, NPAD), jnp.float32)),
    mesh=_mesh,
    compiler_params=_sc_params,
    scratch_types=[
        pltpu.VMEM((NPAD,), jnp.float32),
        pltpu.VMEM((ROWS, COLS), jnp.int32),
        pltpu.VMEM((ROWS, COLS), jnp.int32),
        pltpu.VMEM((CH,), jnp.float32),
        pltpu.VMEM((TILE_N,), jnp.float32),
        pltpu.VMEM((TILE_N,), jnp.float32),
        pltpu.SemaphoreType.DMA,
    ],
)


def _dense_a_body(xT_r, w1_r, b1_r, w2_r, b2_r, w3_r, b3_r,
                  g1_r, gb1_r, g2_r, gb2_r, g3_r, gb3_r,
                  wc1_r, dp_r,
                  g0_r, g1o_r, dinv_r,
                  s1_r, s2_r, s3_r, acc_r, st_r):
    p = pl.program_id(0)
    b = pl.program_id(1)
    eps = 1e-5
    glob = b * BR + lax.broadcasted_iota(jnp.int32, (1, BR), 1)
    msk = glob < N

    @pl.when((p == 0) & (b == 0))
    def _():
        acc_r[...] = jnp.zeros((64, 2), jnp.float32)

    @pl.when(p == 0)
    def _():
        a1 = jnp.dot(w1_r[...], xT_r[...],
                     preferred_element_type=jnp.float32) + b1_r[...]
        s1_r[b] = a1
        a1m = jnp.where(msk, a1, 0.0)
        acc_r[0:8, 0:1] += jnp.sum(a1m, axis=1, keepdims=True)
        acc_r[0:8, 1:2] += jnp.sum(a1m * a1m, axis=1, keepdims=True)

    @pl.when((p == 1) & (b == 0))
    def _():
        m = acc_r[0:8, 0:1] / N
        v = acc_r[0:8, 1:2] / N - m * m
        st_r[0:8, 0:1] = m
        st_r[0:8, 1:2] = lax.rsqrt(v + eps)

    @pl.when(p == 1)
    def _():
        h1 = _lrelu(g1_r[...] * (s1_r[b] - st_r[0:8, 0:1])
                    * st_r[0:8, 1:2] + gb1_r[...])
        a2 = jnp.dot(w2_r[...], h1,
                     preferred_element_type=jnp.float32) + b2_r[...]
        s2_r[b] = a2
        a2m = jnp.where(msk, a2, 0.0)
        acc_r[8:24, 0:1] += jnp.sum(a2m, axis=1, keepdims=True)
        acc_r[8:24, 1:2] += jnp.sum(a2m * a2m, axis=1, keepdims=True)

    @pl.when((p == 2) & (b == 0))
    def _():
        m = acc_r[8:24, 0:1] / N
        v = acc_r[8:24, 1:2] / N - m * m
        st_r[8:24, 0:1] = m
        st_r[8:24, 1:2] = lax.rsqrt(v + eps)

    @pl.when(p == 2)
    def _():
        h2 = _lrelu(g2_r[...] * (s2_r[b] - st_r[8:24, 0:1])
                    * st_r[8:24, 1:2] + gb2_r[...])
        a3 = jnp.dot(w3_r[...], h2,
                     preferred_element_type=jnp.float32) + b3_r[...]
        s3_r[b] = a3
        a3m = jnp.where(msk, a3, 0.0)
        acc_r[24:56, 0:1] += jnp.sum(a3m, axis=1, keepdims=True)
        acc_r[24:56, 1:2] += jnp.sum(a3m * a3m, axis=1, keepdims=True)

    @pl.when((p == 3) & (b == 0))
    def _():
        m = acc_r[24:56, 0:1] / N
        v = acc_r[24:56, 1:2] / N - m * m
        st_r[24:56, 0:1] = m
        st_r[24:56, 1:2] = lax.rsqrt(v + eps)

    @pl.when(p == 3)
    def _():
        h3 = _lrelu(g3_r[...] * (s3_r[b] - st_r[24:56, 0:1])
                    * st_r[24:56, 1:2] + gb3_r[...])
        h1pT = jnp.dot(wc1_r[...], h3, preferred_element_type=jnp.float32)
        deg = dp_r[0:1, :] + dp_r[1:2, :] + 1.0
        dinv = lax.rsqrt(deg)
        gT = dinv * h1pT
        gblk = jnp.transpose(gT)
        g0_r[...] = gblk[:, 0:16]
        g1o_r[...] = gblk[:, 16:32]
        dinv_r[...] = dinv


def _dense_b_body(s0_r, s1_r, g0_r, g1_r,
                  dinv_r, bc1_r, wc2_r, out_r):
    dinv = jnp.transpose(dinv_r[...])
    y2 = jnp.zeros((BRB, 1), jnp.float32)
    for i, (s_r, g_r) in enumerate(((s0_r, g0_r), (s1_r, g1_r))):
        z = _lrelu(dinv * (s_r[...] + g_r[...])
                   + bc1_r[...][:, i * 16:(i + 1) * 16])
        y2 = y2 + jnp.dot(z, wc2_r[...][i * 16:(i + 1) * 16],
                          preferred_element_type=jnp.float32)
    out_r[...] = jnp.transpose(dinv * y2)


def _dense_c_body(tp_r, g2_r, dinv_r, bc2_r, out_r):
    t = tp_r[0:1, :] + tp_r[1:2, :] + g2_r[...]
    out_r[...] = jnp.transpose(dinv_r[...] * t + bc2_r[...])


def kernel(x, edge_index, fc1_W, fc1_b, fc2_W, fc2_b, fc3_W, fc3_b,
           bn1_g, bn1_b, bn2_g, bn2_b, bn3_g, bn3_b,
           conv1_W, conv1_b, conv2_W, conv2_b):
    f32 = jnp.float32
    eiR = edge_index.reshape(2, NT, NCHUNK, ROWS, COLS)

    zcol = jnp.zeros((NPAD,), f32)
    dp, _dpart = _deg(eiR, zcol)

    col = lambda v: v.reshape(-1, 1)
    full = lambda shape: pl.BlockSpec(shape, lambda p, b: (0,) * len(shape))
    g_spec = pl.BlockSpec((BR, F), lambda p, b: (b, 0))
    lane_spec = pl.BlockSpec((2, BR), lambda p, b: (0, b))

    xT = jnp.pad(x.T, ((0, 0), (0, NPAD - N)))
    g0, g1, dinv = pl.pallas_call(
        _dense_a_body,
        grid=(4, NB),
        in_specs=[
            lane_spec,
            full((8, 2)), full((8, 1)),
            full((16, 8)), full((16, 1)),
            full((32, 16)), full((32, 1)),
            full((8, 1)), full((8, 1)),
            full((16, 1)), full((16, 1)),
            full((32, 1)), full((32, 1)),
            full((32, 32)),
            lane_spec,
        ],
        out_specs=[g_spec, g_spec,
                   pl.BlockSpec((1, BR), lambda p, b: (0, b))],
        out_shape=(jax.ShapeDtypeStruct((NPAD, F), f32),
                   jax.ShapeDtypeStruct((NPAD, F), f32),
                   jax.ShapeDtypeStruct((1, NPAD), f32)),
        scratch_shapes=[
            pltpu.VMEM((NB, 8, BR), f32),
            pltpu.VMEM((NB, 16, BR), f32),
            pltpu.VMEM((NB, 32, BR), f32),
            pltpu.VMEM((64, 2), f32),
            pltpu.VMEM((64, 2), f32),
        ],
    )(xT, fc1_W, col(fc1_b), fc2_W, col(fc2_b), fc3_W, col(fc3_b),
      col(bn1_g), col(bn1_b), col(bn2_g), col(bn2_b), col(bn3_g),
      col(bn3_b), conv1_W, dp)

    zrows = jnp.zeros((TILE_N, F), f32)
    s0, s1 = _prop16(eiR, g0, g1, zrows)

    gb_spec = pl.BlockSpec((BRB, F), lambda b: (b, 0))
    row1_spec = pl.BlockSpec((1, BRB), lambda b: (0, b))
    fullb = lambda shape: pl.BlockSpec(shape, lambda b: (0,) * len(shape))

    g2 = pl.pallas_call(
        _dense_b_body,
        grid=(NBB,),
        in_specs=[gb_spec, gb_spec, gb_spec, gb_spec,
                  row1_spec, fullb((1, 32)), fullb((32, 1))],
        out_specs=row1_spec,
        out_shape=jax.ShapeDtypeStruct((1, NPAD), f32),
    )(s0, s1, g0, g1, dinv,
      conv1_b.reshape(1, 32), conv2_W.T)

    tp, _tpart = _prop1(eiR, g2.reshape(NPAD), zcol)

    out = pl.pallas_call(
        _dense_c_body,
        grid=(NBB,),
        in_specs=[pl.BlockSpec((2, BRB), lambda b: (0, b)),
                  row1_spec, row1_spec, fullb((1, 1))],
        out_specs=pl.BlockSpec((BRB, 1), lambda b: (b, 0)),
        out_shape=jax.ShapeDtypeStruct((NPAD, 1), f32),
    )(tp, g2, dinv, conv2_b.reshape(1, 1))
    return out[:N]


# speedup vs baseline: 51.5438x; 1.0087x over previous
import jax
import jax.numpy as jnp
from jax import lax
from jax.experimental import pallas as pl
from jax.experimental.pallas import tpu as pltpu
from jax.experimental.pallas import tpu_sc as plsc

N = 100000
E = 1600000
NT = 16
NC = 2
CH = 400
NCHUNK = E // NT // CH
NPAD = 100352
TILE_N = NPAD // NT
F = 16
BR = 6272
NB = NPAD // BR
BRB = 2048
NBB = NPAD // BRB

_mesh = plsc.VectorSubcoreMesh(core_axis_name="c", subcore_axis_name="s")
_sc_params = pltpu.CompilerParams(needs_layout_passes=False,
                                  use_tc_tiling_on_sc=False)


def _lrelu(h):
    return jnp.where(h >= 0, h, 0.1 * h)


def _reduce_tiles(src_vmem, part_hbm, out_hbm, c, s, accbuf, tbuf):
    base = s * TILE_N
    pltpu.sync_copy(src_vmem, part_hbm.at[c, s])
    plsc.subcore_barrier()

    def addloop(i, _):
        for u in range(8):
            sl = pl.ds(i * 128 + u * 16, 16)
            accbuf[sl] = accbuf[sl] + tbuf[sl]
        return 0

    pltpu.sync_copy(part_hbm.at[c, 0, pl.ds(base, TILE_N)], accbuf)
    for t in range(1, NT):
        pltpu.sync_copy(part_hbm.at[c, t, pl.ds(base, TILE_N)], tbuf)
        lax.fori_loop(0, TILE_N // 128, addloop, 0)
    pltpu.sync_copy(accbuf, out_hbm.at[c, pl.ds(base, TILE_N)])


def _deg_body(eiF, zcol, dp, dpart, part, ibuf, accbuf, tbuf):
    c = lax.axis_index("c")
    s = lax.axis_index("s")
    ones = jnp.ones((16,), jnp.float32)
    pltpu.sync_copy(zcol, part)

    def chunk(k, _):
        pltpu.sync_copy(eiF.at[c, s, k], ibuf)

        def iloop(i, _):
            idx = ibuf[pl.ds(i * 16, 16)]
            plsc.addupdate_scatter(part, [idx], ones)
            return 0
        lax.fori_loop(0, CH // 16, iloop, 0)
        return 0
    lax.fori_loop(0, NCHUNK, chunk, 0)

    _reduce_tiles(part, dpart, dp, c, s, accbuf, tbuf)


_deg = pl.kernel(
    _deg_body,
    out_type=(jax.ShapeDtypeStruct((NC, NPAD), jnp.float32),
              jax.ShapeDtypeStruct((NC, NT, NPAD), jnp.float32)),
    mesh=_mesh,
    compiler_params=_sc_params,
    scratch_types=[
        pltpu.VMEM((NPAD,), jnp.float32),
        pltpu.VMEM((CH,), jnp.int32),
        pltpu.VMEM((TILE_N,), jnp.float32),
        pltpu.VMEM((TILE_N,), jnp.float32),
    ],
)


def _prop16_body(eiF, gLo, gHi, zrows, outLo, outHi, sidx3, didx3, rows2,
                 acc, isem, gsem, ssem):
    c = lax.axis_index("c")
    s = lax.axis_index("s")

    base = s * TILE_N
    pltpu.sync_copy(zrows, acc.at[pl.ds(base, TILE_N)])
    plsc.subcore_barrier()

    def fetch_idx(d, k, kb):
        pltpu.async_copy(eiF.at[d, s, k], sidx3.at[kb], isem)
        pltpu.async_copy(eiF.at[1 - d, s, k], didx3.at[kb], isem)

    def wait_idx():
        pltpu.make_async_copy(eiF.at[0, s, 0], sidx3.at[0], isem).wait()
        pltpu.make_async_copy(eiF.at[0, s, 0], didx3.at[0], isem).wait()

    def drain_scat():
        pltpu.make_async_copy(rows2.at[0], acc.at[didx3.at[0]], ssem).wait()

    for d in range(2):
        fetch_idx(d, 0, 0)

        def chunk(k, _, d=d):
            ki = lax.rem(k, 3)
            kr = lax.rem(k, 2)

            @pl.when(k >= 2)
            def _():
                drain_scat()
            wait_idx()

            @pl.when(k < NCHUNK - 1)
            def _():
                fetch_idx(d, k + 1, lax.rem(k + 1, 3))

            @pl.when(c == 0)
            def _():
                pltpu.async_copy(gLo.at[sidx3.at[ki]], rows2.at[kr],
                                 gsem).wait()

            @pl.when(c == 1)
            def _():
                pltpu.async_copy(gHi.at[sidx3.at[ki]], rows2.at[kr],
                                 gsem).wait()
            pltpu.async_copy(rows2.at[kr], acc.at[didx3.at[ki]], ssem,
                             add=True)
            return 0
        lax.fori_loop(0, NCHUNK, chunk, 0)
        drain_scat()
        drain_scat()

    plsc.subcore_barrier()

    @pl.when(c == 0)
    def _():
        for i in range(4):
            pltpu.sync_copy(acc.at[pl.ds(base + i * 1568, 1568)],
                            outLo.at[pl.ds(base + i * 1568, 1568)])

    @pl.when(c == 1)
    def _():
        for i in range(4):
            pltpu.sync_copy(acc.at[pl.ds(base + i * 1568, 1568)],
                            outHi.at[pl.ds(base + i * 1568, 1568)])


_prop16 = pl.kernel(
    _prop16_body,
    out_type=(jax.ShapeDtypeStruct((NPAD, F), jnp.float32),
              jax.ShapeDtypeStruct((NPAD, F), jnp.float32)),
    mesh=_mesh,
    compiler_params=_sc_params,
    scratch_types=[
        pltpu.VMEM((3, CH), jnp.int32),
        pltpu.VMEM((3, CH), jnp.int32),
        pltpu.VMEM((2, CH, F), jnp.float32),
        pltpu.VMEM_SHARED((NPAD, F), jnp.float32),
        pltpu.SemaphoreType.DMA,
        pltpu.SemaphoreType.DMA,
        pltpu.SemaphoreType.DMA,
    ],
)


def _prop1_body(eiF, g2, zcol, tp, tpart, part, sidx, dbuf, vbuf, accbuf,
                tbuf, sem):
    c = lax.axis_index("c")
    s = lax.axis_index("s")
    pltpu.sync_copy(zcol, part)

    def chunk(k, _):
        pltpu.sync_copy(eiF.at[c, s, k], sidx)
        pltpu.sync_copy(eiF.at[1 - c, s, k], dbuf)
        pltpu.async_copy(g2.at[sidx], vbuf, sem).wait()

        def iloop(i, _):
            sl = pl.ds(i * 16, 16)
            plsc.addupdate_scatter(part, [dbuf[sl]], vbuf[sl])
            return 0
        lax.fori_loop(0, CH // 16, iloop, 0)
        return 0
    lax.fori_loop(0, NCHUNK, chunk, 0)

    _reduce_tiles(part, tpart, tp, c, s, accbuf, tbuf)


_prop1 = pl.kernel(
    _prop1_body,
    out_type=(jax.ShapeDtypeStruct((NC, NPAD), jnp.float32),
              jax.ShapeDtypeStruct((NC, NT, NPAD), jnp.float32)),
    mesh=_mesh,
    compiler_params=_sc_params,
    scratch_types=[
        pltpu.VMEM((NPAD,), jnp.float32),
        pltpu.VMEM((CH,), jnp.int32),
        pltpu.VMEM((CH,), jnp.int32),
        pltpu.VMEM((CH,), jnp.float32),
        pltpu.VMEM((TILE_N,), jnp.float32),
        pltpu.VMEM((TILE_N,), jnp.float32),
        pltpu.SemaphoreType.DMA,
    ],
)


def _dense_a_body(xT_r, w1_r, b1_r, w2_r, b2_r, w3_r, b3_r,
                  g1_r, gb1_r, g2_r, gb2_r, g3_r, gb3_r,
                  wc1_r, dp_r,
                  g0_r, g1o_r, dinv_r,
                  s1_r, s2_r, s3_r, acc_r, st_r):
    p = pl.program_id(0)
    b = pl.program_id(1)
    eps = 1e-5
    glob = b * BR + lax.broadcasted_iota(jnp.int32, (1, BR), 1)
    msk = glob < N

    @pl.when((p == 0) & (b == 0))
    def _():
        acc_r[...] = jnp.zeros((64, 2), jnp.float32)

    @pl.when(p == 0)
    def _():
        a1 = jnp.dot(w1_r[...], xT_r[...],
                     preferred_element_type=jnp.float32) + b1_r[...]
        s1_r[b] = a1
        a1m = jnp.where(msk, a1, 0.0)
        acc_r[0:8, 0:1] += jnp.sum(a1m, axis=1, keepdims=True)
        acc_r[0:8, 1:2] += jnp.sum(a1m * a1m, axis=1, keepdims=True)

    @pl.when((p == 1) & (b == 0))
    def _():
        m = acc_r[0:8, 0:1] / N
        v = acc_r[0:8, 1:2] / N - m * m
        st_r[0:8, 0:1] = m
        st_r[0:8, 1:2] = lax.rsqrt(v + eps)

    @pl.when(p == 1)
    def _():
        h1 = _lrelu(g1_r[...] * (s1_r[b] - st_r[0:8, 0:1])
                    * st_r[0:8, 1:2] + gb1_r[...])
        a2 = jnp.dot(w2_r[...], h1,
                     preferred_element_type=jnp.float32) + b2_r[...]
        s2_r[b] = a2
        a2m = jnp.where(msk, a2, 0.0)
        acc_r[8:24, 0:1] += jnp.sum(a2m, axis=1, keepdims=True)
        acc_r[8:24, 1:2] += jnp.sum(a2m * a2m, axis=1, keepdims=True)

    @pl.when((p == 2) & (b == 0))
    def _():
        m = acc_r[8:24, 0:1] / N
        v = acc_r[8:24, 1:2] / N - m * m
        st_r[8:24, 0:1] = m
        st_r[8:24, 1:2] = lax.rsqrt(v + eps)

    @pl.when(p == 2)
    def _():
        h2 = _lrelu(g2_r[...] * (s2_r[b] - st_r[8:24, 0:1])
                    * st_r[8:24, 1:2] + gb2_r[...])
        a3 = jnp.dot(w3_r[...], h2,
                     preferred_element_type=jnp.float32) + b3_r[...]
        s3_r[b] = a3
        a3m = jnp.where(msk, a3, 0.0)
        acc_r[24:56, 0:1] += jnp.sum(a3m, axis=1, keepdims=True)
        acc_r[24:56, 1:2] += jnp.sum(a3m * a3m, axis=1, keepdims=True)

    @pl.when((p == 3) & (b == 0))
    def _():
        m = acc_r[24:56, 0:1] / N
        v = acc_r[24:56, 1:2] / N - m * m
        st_r[24:56, 0:1] = m
        st_r[24:56, 1:2] = lax.rsqrt(v + eps)

    @pl.when(p == 3)
    def _():
        h3 = _lrelu(g3_r[...] * (s3_r[b] - st_r[24:56, 0:1])
                    * st_r[24:56, 1:2] + gb3_r[...])
        h1pT = jnp.dot(wc1_r[...], h3, preferred_element_type=jnp.float32)
        deg = dp_r[0:1, :] + dp_r[1:2, :] + 1.0
        dinv = lax.rsqrt(deg)
        gT = dinv * h1pT
        gblk = jnp.transpose(gT)
        g0_r[...] = gblk[:, 0:16]
        g1o_r[...] = gblk[:, 16:32]
        dinv_r[...] = dinv


def _dense_b_body(s0_r, s1_r, g0_r, g1_r,
                  dinv_r, bc1_r, wc2_r, out_r):
    dinv = jnp.transpose(dinv_r[...])
    y2 = jnp.zeros((BRB, 1), jnp.float32)
    for i, (s_r, g_r) in enumerate(((s0_r, g0_r), (s1_r, g1_r))):
        z = _lrelu(dinv * (s_r[...] + g_r[...])
                   + bc1_r[...][:, i * 16:(i + 1) * 16])
        y2 = y2 + jnp.dot(z, wc2_r[...][i * 16:(i + 1) * 16],
                          preferred_element_type=jnp.float32)
    out_r[...] = jnp.transpose(dinv * y2)


def _dense_c_body(tp_r, g2_r, dinv_r, bc2_r, out_r):
    t = tp_r[0:1, :] + tp_r[1:2, :] + g2_r[...]
    out_r[...] = jnp.transpose(dinv_r[...] * t + bc2_r[...])


def kernel(x, edge_index, fc1_W, fc1_b, fc2_W, fc2_b, fc3_W, fc3_b,
           bn1_g, bn1_b, bn2_g, bn2_b, bn3_g, bn3_b,
           conv1_W, conv1_b, conv2_W, conv2_b):
    f32 = jnp.float32
    eiF = edge_index.reshape(2, NT, NCHUNK, CH)

    zcol = jnp.zeros((NPAD,), f32)
    dp, _dpart = _deg(eiF, zcol)

    col = lambda v: v.reshape(-1, 1)
    full = lambda shape: pl.BlockSpec(shape, lambda p, b: (0,) * len(shape))
    g_spec = pl.BlockSpec((BR, F), lambda p, b: (b, 0))
    lane_spec = pl.BlockSpec((2, BR), lambda p, b: (0, b))

    xT = jnp.pad(x.T, ((0, 0), (0, NPAD - N)))
    g0, g1, dinv = pl.pallas_call(
        _dense_a_body,
        grid=(4, NB),
        in_specs=[
            lane_spec,
            full((8, 2)), full((8, 1)),
            full((16, 8)), full((16, 1)),
            full((32, 16)), full((32, 1)),
            full((8, 1)), full((8, 1)),
            full((16, 1)), full((16, 1)),
            full((32, 1)), full((32, 1)),
            full((32, 32)),
            lane_spec,
        ],
        out_specs=[g_spec, g_spec,
                   pl.BlockSpec((1, BR), lambda p, b: (0, b))],
        out_shape=(jax.ShapeDtypeStruct((NPAD, F), f32),
                   jax.ShapeDtypeStruct((NPAD, F), f32),
                   jax.ShapeDtypeStruct((1, NPAD), f32)),
        scratch_shapes=[
            pltpu.VMEM((NB, 8, BR), f32),
            pltpu.VMEM((NB, 16, BR), f32),
            pltpu.VMEM((NB, 32, BR), f32),
            pltpu.VMEM((64, 2), f32),
            pltpu.VMEM((64, 2), f32),
        ],
    )(xT, fc1_W, col(fc1_b), fc2_W, col(fc2_b), fc3_W, col(fc3_b),
      col(bn1_g), col(bn1_b), col(bn2_g), col(bn2_b), col(bn3_g),
      col(bn3_b), conv1_W, dp)

    zrows = jnp.zeros((TILE_N, F), f32)
    s0, s1 = _prop16(eiF, g0, g1, zrows)

    gb_spec = pl.BlockSpec((BRB, F), lambda b: (b, 0))
    row1_spec = pl.BlockSpec((1, BRB), lambda b: (0, b))
    fullb = lambda shape: pl.BlockSpec(shape, lambda b: (0,) * len(shape))

    g2 = pl.pallas_call(
        _dense_b_body,
        grid=(NBB,),
        in_specs=[gb_spec, gb_spec, gb_spec, gb_spec,
                  row1_spec, fullb((1, 32)), fullb((32, 1))],
        out_specs=row1_spec,
        out_shape=jax.ShapeDtypeStruct((1, NPAD), f32),
    )(s0, s1, g0, g1, dinv,
      conv1_b.reshape(1, 32), conv2_W.T)

    tp, _tpart = _prop1(eiF, g2.reshape(NPAD), zcol)

    out = pl.pallas_call(
        _dense_c_body,
        grid=(NBB,),
        in_specs=[pl.BlockSpec((2, BRB), lambda b: (0, b)),
                  row1_spec, row1_spec, fullb((1, 1))],
        out_specs=pl.BlockSpec((BRB, 1), lambda b: (b, 0)),
        out_shape=jax.ShapeDtypeStruct((NPAD, 1), f32),
    )(tp, g2, dinv, conv2_b.reshape(1, 1))
    return out[:N]


# speedup vs baseline: 74.9364x; 1.4538x over previous
import jax
import jax.numpy as jnp
from jax import lax
from jax.experimental import pallas as pl
from jax.experimental.pallas import tpu as pltpu
from jax.experimental.pallas import tpu_sc as plsc

N = 100000
E = 1600000
NT = 16
NC = 2
CH = 400
NCHUNK = E // NT // CH
MB = 10
NMB = NCHUNK // MB
NPAD = 100352
TILE_N = NPAD // NT
F = 16
BR = 6272
NB = NPAD // BR
BRB = 2048
NBB = NPAD // BRB

_mesh = plsc.VectorSubcoreMesh(core_axis_name="c", subcore_axis_name="s")
_sc_params = pltpu.CompilerParams(needs_layout_passes=False,
                                  use_tc_tiling_on_sc=False)


def _lrelu(h):
    return jnp.where(h >= 0, h, 0.1 * h)


def _reduce_tiles(src_vmem, part_hbm, out_hbm, c, s, accbuf, tbuf):
    base = s * TILE_N
    pltpu.sync_copy(src_vmem, part_hbm.at[c, s])
    plsc.subcore_barrier()

    def addloop(i, _):
        for u in range(8):
            sl = pl.ds(i * 128 + u * 16, 16)
            accbuf[sl] = accbuf[sl] + tbuf[sl]
        return 0

    pltpu.sync_copy(part_hbm.at[c, 0, pl.ds(base, TILE_N)], accbuf)
    for t in range(1, NT):
        pltpu.sync_copy(part_hbm.at[c, t, pl.ds(base, TILE_N)], tbuf)
        lax.fori_loop(0, TILE_N // 128, addloop, 0)
    pltpu.sync_copy(accbuf, out_hbm.at[c, pl.ds(base, TILE_N)])


def _deg_body(eiF, zcol, dp, dpart, part, ibuf, accbuf, tbuf):
    c = lax.axis_index("c")
    s = lax.axis_index("s")
    ones = jnp.ones((16,), jnp.float32)
    pltpu.sync_copy(zcol, part)

    def chunk(k, _):
        pltpu.sync_copy(eiF.at[c, s, pl.ds(k * MB, MB)], ibuf)

        def mloop(m, _):
            def iloop(i, _):
                idx = ibuf[m, pl.ds(i * 16, 16)]
                plsc.addupdate_scatter(part, [idx], ones)
                return 0
            lax.fori_loop(0, CH // 16, iloop, 0)
            return 0
        lax.fori_loop(0, MB, mloop, 0)
        return 0
    lax.fori_loop(0, NMB, chunk, 0)

    _reduce_tiles(part, dpart, dp, c, s, accbuf, tbuf)


_deg = pl.kernel(
    _deg_body,
    out_type=(jax.ShapeDtypeStruct((NC, NPAD), jnp.float32),
              jax.ShapeDtypeStruct((NC, NT, NPAD), jnp.float32)),
    mesh=_mesh,
    compiler_params=_sc_params,
    scratch_types=[
        pltpu.VMEM((NPAD,), jnp.float32),
        pltpu.VMEM((MB, CH), jnp.int32),
        pltpu.VMEM((TILE_N,), jnp.float32),
        pltpu.VMEM((TILE_N,), jnp.float32),
    ],
)


def _prop16_body(eiF, gLo, gHi, zrows, outLo, outHi, sidx3, didx3, rows2,
                 acc, isem, gsem, ssem):
    c = lax.axis_index("c")
    s = lax.axis_index("s")

    base = s * TILE_N
    pltpu.sync_copy(zrows, acc.at[pl.ds(base, TILE_N)])
    plsc.subcore_barrier()

    def fetch_idx(d, k, kb):
        pltpu.async_copy(eiF.at[d, s, k], sidx3.at[kb], isem)
        pltpu.async_copy(eiF.at[1 - d, s, k], didx3.at[kb], isem)

    def wait_idx():
        pltpu.make_async_copy(eiF.at[0, s, 0], sidx3.at[0], isem).wait()
        pltpu.make_async_copy(eiF.at[0, s, 0], didx3.at[0], isem).wait()

    def drain_scat():
        pltpu.make_async_copy(rows2.at[0], acc.at[didx3.at[0]], ssem).wait()

    def wait_gather(g):
        pltpu.make_async_copy(g.at[sidx3.at[0]], rows2.at[0], gsem).wait()

    def fire_scatter(ki, kr):
        pltpu.async_copy(rows2.at[kr], acc.at[didx3.at[ki]], ssem, add=True)

    for d in range(2):
        fetch_idx(d, 0, 0)

        def chunk(k, _, d=d):
            ki = lax.rem(k, 3)
            kr = lax.rem(k, 3)

            @pl.when(k >= 3)
            def _():
                drain_scat()
            wait_idx()

            @pl.when(k < NCHUNK - 1)
            def _():
                fetch_idx(d, k + 1, lax.rem(k + 1, 3))

            @pl.when(c == 0)
            def _():
                pltpu.async_copy(gLo.at[sidx3.at[ki]], rows2.at[kr], gsem)

            @pl.when(c == 1)
            def _():
                pltpu.async_copy(gHi.at[sidx3.at[ki]], rows2.at[kr], gsem)

            @pl.when(k >= 1)
            def _():
                wait_gather(gLo)
                fire_scatter(lax.rem(k - 1, 3), lax.rem(k - 1, 3))
            return 0
        lax.fori_loop(0, NCHUNK, chunk, 0)
        wait_gather(gLo)
        fire_scatter(lax.rem(NCHUNK - 1, 3), lax.rem(NCHUNK - 1, 3))
        drain_scat()
        drain_scat()
        drain_scat()

    plsc.subcore_barrier()

    @pl.when(c == 0)
    def _():
        for i in range(4):
            pltpu.sync_copy(acc.at[pl.ds(base + i * 1568, 1568)],
                            outLo.at[pl.ds(base + i * 1568, 1568)])

    @pl.when(c == 1)
    def _():
        for i in range(4):
            pltpu.sync_copy(acc.at[pl.ds(base + i * 1568, 1568)],
                            outHi.at[pl.ds(base + i * 1568, 1568)])


_prop16 = pl.kernel(
    _prop16_body,
    out_type=(jax.ShapeDtypeStruct((NPAD, F), jnp.float32),
              jax.ShapeDtypeStruct((NPAD, F), jnp.float32)),
    mesh=_mesh,
    compiler_params=_sc_params,
    scratch_types=[
        pltpu.VMEM((3, CH), jnp.int32),
        pltpu.VMEM((3, CH), jnp.int32),
        pltpu.VMEM((3, CH, F), jnp.float32),
        pltpu.VMEM_SHARED((NPAD, F), jnp.float32),
        pltpu.SemaphoreType.DMA,
        pltpu.SemaphoreType.DMA,
        pltpu.SemaphoreType.DMA,
    ],
)


def _prop1_body(eiF, g2, zcol, tp, tpart, part, sidx, dbuf, vbuf, accbuf,
                tbuf, sem):
    c = lax.axis_index("c")
    s = lax.axis_index("s")
    pltpu.sync_copy(zcol, part)

    def chunk(k, _):
        pltpu.sync_copy(eiF.at[c, s, pl.ds(k * MB, MB)], sidx)
        pltpu.sync_copy(eiF.at[1 - c, s, pl.ds(k * MB, MB)], dbuf)
        for m in range(MB):
            pltpu.async_copy(g2.at[sidx.at[m]], vbuf.at[m], sem)
        for m in range(MB):
            pltpu.make_async_copy(g2.at[sidx.at[0]], vbuf.at[0], sem).wait()

            def iloop(i, _, m=m):
                sl = pl.ds(i * 16, 16)
                plsc.addupdate_scatter(part, [dbuf[m, sl]],
                                       vbuf[m, sl])
                return 0
            lax.fori_loop(0, CH // 16, iloop, 0)
        return 0
    lax.fori_loop(0, NMB, chunk, 0)

    _reduce_tiles(part, tpart, tp, c, s, accbuf, tbuf)


_prop1 = pl.kernel(
    _prop1_body,
    out_type=(jax.ShapeDtypeStruct((NC, NPAD), jnp.float32),
              jax.ShapeDtypeStruct((NC, NT, NPAD), jnp.float32)),
    mesh=_mesh,
    compiler_params=_sc_params,
    scratch_types=[
        pltpu.VMEM((NPAD,), jnp.float32),
        pltpu.VMEM((MB, CH), jnp.int32),
        pltpu.VMEM((MB, CH), jnp.int32),
        pltpu.VMEM((MB, CH), jnp.float32),
        pltpu.VMEM((TILE_N,), jnp.float32),
        pltpu.VMEM((TILE_N,), jnp.float32),
        pltpu.SemaphoreType.DMA,
    ],
)


def _dense_a_body(xT_r, w1_r, b1_r, w2_r, b2_r, w3_r, b3_r,
                  g1_r, gb1_r, g2_r, gb2_r, g3_r, gb3_r,
                  wc1_r, dp_r,
                  g0_r, g1o_r, dinv_r,
                  s1_r, s2_r, s3_r, acc_r, st_r):
    p = pl.program_id(0)
    b = pl.program_id(1)
    eps = 1e-5
    glob = b * BR + lax.broadcasted_iota(jnp.int32, (1, BR), 1)
    msk = glob < N

    @pl.when((p == 0) & (b == 0))
    def _():
        acc_r[...] = jnp.zeros((64, 2), jnp.float32)

    @pl.when(p == 0)
    def _():
        a1 = jnp.dot(w1_r[...], xT_r[...],
                     preferred_element_type=jnp.float32) + b1_r[...]
        s1_r[b] = a1
        a1m = jnp.where(msk, a1, 0.0)
        acc_r[0:8, 0:1] += jnp.sum(a1m, axis=1, keepdims=True)
        acc_r[0:8, 1:2] += jnp.sum(a1m * a1m, axis=1, keepdims=True)

    @pl.when((p == 1) & (b == 0))
    def _():
        m = acc_r[0:8, 0:1] / N
        v = acc_r[0:8, 1:2] / N - m * m
        st_r[0:8, 0:1] = m
        st_r[0:8, 1:2] = lax.rsqrt(v + eps)

    @pl.when(p == 1)
    def _():
        h1 = _lrelu(g1_r[...] * (s1_r[b] - st_r[0:8, 0:1])
                    * st_r[0:8, 1:2] + gb1_r[...])
        a2 = jnp.dot(w2_r[...], h1,
                     preferred_element_type=jnp.float32) + b2_r[...]
        s2_r[b] = a2
        a2m = jnp.where(msk, a2, 0.0)
        acc_r[8:24, 0:1] += jnp.sum(a2m, axis=1, keepdims=True)
        acc_r[8:24, 1:2] += jnp.sum(a2m * a2m, axis=1, keepdims=True)

    @pl.when((p == 2) & (b == 0))
    def _():
        m = acc_r[8:24, 0:1] / N
        v = acc_r[8:24, 1:2] / N - m * m
        st_r[8:24, 0:1] = m
        st_r[8:24, 1:2] = lax.rsqrt(v + eps)

    @pl.when(p == 2)
    def _():
        h2 = _lrelu(g2_r[...] * (s2_r[b] - st_r[8:24, 0:1])
                    * st_r[8:24, 1:2] + gb2_r[...])
        a3 = jnp.dot(w3_r[...], h2,
                     preferred_element_type=jnp.float32) + b3_r[...]
        s3_r[b] = a3
        a3m = jnp.where(msk, a3, 0.0)
        acc_r[24:56, 0:1] += jnp.sum(a3m, axis=1, keepdims=True)
        acc_r[24:56, 1:2] += jnp.sum(a3m * a3m, axis=1, keepdims=True)

    @pl.when((p == 3) & (b == 0))
    def _():
        m = acc_r[24:56, 0:1] / N
        v = acc_r[24:56, 1:2] / N - m * m
        st_r[24:56, 0:1] = m
        st_r[24:56, 1:2] = lax.rsqrt(v + eps)

    @pl.when(p == 3)
    def _():
        h3 = _lrelu(g3_r[...] * (s3_r[b] - st_r[24:56, 0:1])
                    * st_r[24:56, 1:2] + gb3_r[...])
        h1pT = jnp.dot(wc1_r[...], h3, preferred_element_type=jnp.float32)
        deg = dp_r[0:1, :] + dp_r[1:2, :] + 1.0
        dinv = lax.rsqrt(deg)
        gT = dinv * h1pT
        gblk = jnp.transpose(gT)
        g0_r[...] = gblk[:, 0:16]
        g1o_r[...] = gblk[:, 16:32]
        dinv_r[...] = dinv


def _dense_b_body(s0_r, s1_r, g0_r, g1_r,
                  dinv_r, bc1_r, wc2_r, out_r):
    dinv = jnp.transpose(dinv_r[...])
    y2 = jnp.zeros((BRB, 1), jnp.float32)
    for i, (s_r, g_r) in enumerate(((s0_r, g0_r), (s1_r, g1_r))):
        z = _lrelu(dinv * (s_r[...] + g_r[...])
                   + bc1_r[...][:, i * 16:(i + 1) * 16])
        y2 = y2 + jnp.dot(z, wc2_r[...][i * 16:(i + 1) * 16],
                          preferred_element_type=jnp.float32)
    out_r[...] = jnp.transpose(dinv * y2)


def _dense_c_body(tp_r, g2_r, dinv_r, bc2_r, out_r):
    t = tp_r[0:1, :] + tp_r[1:2, :] + g2_r[...]
    out_r[...] = jnp.transpose(dinv_r[...] * t + bc2_r[...])


def kernel(x, edge_index, fc1_W, fc1_b, fc2_W, fc2_b, fc3_W, fc3_b,
           bn1_g, bn1_b, bn2_g, bn2_b, bn3_g, bn3_b,
           conv1_W, conv1_b, conv2_W, conv2_b):
    f32 = jnp.float32
    eiF = edge_index.reshape(2, NT, NCHUNK, CH)

    zcol = jnp.zeros((NPAD,), f32)
    dp, _dpart = _deg(eiF, zcol)

    col = lambda v: v.reshape(-1, 1)
    full = lambda shape: pl.BlockSpec(shape, lambda p, b: (0,) * len(shape))
    g_spec = pl.BlockSpec((BR, F), lambda p, b: (b, 0))
    lane_spec = pl.BlockSpec((2, BR), lambda p, b: (0, b))

    xT = jnp.pad(x.T, ((0, 0), (0, NPAD - N)))
    g0, g1, dinv = pl.pallas_call(
        _dense_a_body,
        grid=(4, NB),
        in_specs=[
            lane_spec,
            full((8, 2)), full((8, 1)),
            full((16, 8)), full((16, 1)),
            full((32, 16)), full((32, 1)),
            full((8, 1)), full((8, 1)),
            full((16, 1)), full((16, 1)),
            full((32, 1)), full((32, 1)),
            full((32, 32)),
            lane_spec,
        ],
        out_specs=[g_spec, g_spec,
                   pl.BlockSpec((1, BR), lambda p, b: (0, b))],
        out_shape=(jax.ShapeDtypeStruct((NPAD, F), f32),
                   jax.ShapeDtypeStruct((NPAD, F), f32),
                   jax.ShapeDtypeStruct((1, NPAD), f32)),
        scratch_shapes=[
            pltpu.VMEM((NB, 8, BR), f32),
            pltpu.VMEM((NB, 16, BR), f32),
            pltpu.VMEM((NB, 32, BR), f32),
            pltpu.VMEM((64, 2), f32),
            pltpu.VMEM((64, 2), f32),
        ],
    )(xT, fc1_W, col(fc1_b), fc2_W, col(fc2_b), fc3_W, col(fc3_b),
      col(bn1_g), col(bn1_b), col(bn2_g), col(bn2_b), col(bn3_g),
      col(bn3_b), conv1_W, dp)

    zrows = jnp.zeros((TILE_N, F), f32)
    s0, s1 = _prop16(eiF, g0, g1, zrows)

    gb_spec = pl.BlockSpec((BRB, F), lambda b: (b, 0))
    row1_spec = pl.BlockSpec((1, BRB), lambda b: (0, b))
    fullb = lambda shape: pl.BlockSpec(shape, lambda b: (0,) * len(shape))

    g2 = pl.pallas_call(
        _dense_b_body,
        grid=(NBB,),
        in_specs=[gb_spec, gb_spec, gb_spec, gb_spec,
                  row1_spec, fullb((1, 32)), fullb((32, 1))],
        out_specs=row1_spec,
        out_shape=jax.ShapeDtypeStruct((1, NPAD), f32),
    )(s0, s1, g0, g1, dinv,
      conv1_b.reshape(1, 32), conv2_W.T)

    tp, _tpart = _prop1(eiF, g2.reshape(NPAD), zcol)

    out = pl.pallas_call(
        _dense_c_body,
        grid=(NBB,),
        in_specs=[pl.BlockSpec((2, BRB), lambda b: (0, b)),
                  row1_spec, row1_spec, fullb((1, 1))],
        out_specs=pl.BlockSpec((BRB, 1), lambda b: (b, 0)),
        out_shape=jax.ShapeDtypeStruct((NPAD, 1), f32),
    )(tp, g2, dinv, conv2_b.reshape(1, 1))
    return out[:N]


# speedup vs baseline: 80.6776x; 1.0766x over previous
import jax
import jax.numpy as jnp
from jax import lax
from jax.experimental import pallas as pl
from jax.experimental.pallas import tpu as pltpu
from jax.experimental.pallas import tpu_sc as plsc

N = 100000
E = 1600000
NT = 16
NC = 2
CH = 400
NCHUNK = E // NT // CH
MB = 5
NMB = NCHUNK // MB
MBD = 25
NMBD = NCHUNK // MBD
NPAD = 100352
TILE_N = NPAD // NT
F = 16
BR = 6272
NB = NPAD // BR
BRB = 2048
NBB = NPAD // BRB

_mesh = plsc.VectorSubcoreMesh(core_axis_name="c", subcore_axis_name="s")
_sc_params = pltpu.CompilerParams(needs_layout_passes=False,
                                  use_tc_tiling_on_sc=False)


def _lrelu(h):
    return jnp.where(h >= 0, h, 0.1 * h)


def _reduce_tiles(src_vmem, part_hbm, out_hbm, c, s, accbuf, tbuf):
    base = s * TILE_N
    pltpu.sync_copy(src_vmem, part_hbm.at[c, s])
    plsc.subcore_barrier()

    def addloop(i, _):
        for u in range(8):
            sl = pl.ds(i * 128 + u * 16, 16)
            accbuf[sl] = accbuf[sl] + tbuf[sl]
        return 0

    pltpu.sync_copy(part_hbm.at[c, 0, pl.ds(base, TILE_N)], accbuf)
    for t in range(1, NT):
        pltpu.sync_copy(part_hbm.at[c, t, pl.ds(base, TILE_N)], tbuf)
        lax.fori_loop(0, TILE_N // 128, addloop, 0)
    pltpu.sync_copy(accbuf, out_hbm.at[c, pl.ds(base, TILE_N)])


def _deg_body(eiF, zcol, dp, dpart, part, ibuf, accbuf, tbuf):
    c = lax.axis_index("c")
    s = lax.axis_index("s")
    ones = jnp.ones((16,), jnp.float32)
    pltpu.sync_copy(zcol, part)

    def chunk(k, _):
        pltpu.sync_copy(eiF.at[c, s, pl.ds(k * MBD, MBD)], ibuf)

        def mloop(m, _):
            def iloop(i, _):
                idx = ibuf[m, pl.ds(i * 16, 16)]
                plsc.addupdate_scatter(part, [idx], ones)
                return 0
            lax.fori_loop(0, CH // 16, iloop, 0)
            return 0
        lax.fori_loop(0, MBD, mloop, 0)
        return 0
    lax.fori_loop(0, NMBD, chunk, 0)

    _reduce_tiles(part, dpart, dp, c, s, accbuf, tbuf)


_deg = pl.kernel(
    _deg_body,
    out_type=(jax.ShapeDtypeStruct((NC, NPAD), jnp.float32),
              jax.ShapeDtypeStruct((NC, NT, NPAD), jnp.float32)),
    mesh=_mesh,
    compiler_params=_sc_params,
    scratch_types=[
        pltpu.VMEM((NPAD,), jnp.float32),
        pltpu.VMEM((MBD, CH), jnp.int32),
        pltpu.VMEM((TILE_N,), jnp.float32),
        pltpu.VMEM((TILE_N,), jnp.float32),
    ],
)


def _prop16_body(eiF, gLo, gHi, zrows, outLo, outHi, sidx3, didx3, rows2,
                 acc, isem, gsem, ssem):
    c = lax.axis_index("c")
    s = lax.axis_index("s")

    base = s * TILE_N
    pltpu.sync_copy(zrows, acc.at[pl.ds(base, TILE_N)])
    plsc.subcore_barrier()

    def fetch_idx(d, k, kb):
        pltpu.async_copy(eiF.at[d, s, k], sidx3.at[kb], isem)
        pltpu.async_copy(eiF.at[1 - d, s, k], didx3.at[kb], isem)

    def wait_idx():
        pltpu.make_async_copy(eiF.at[0, s, 0], sidx3.at[0], isem).wait()
        pltpu.make_async_copy(eiF.at[0, s, 0], didx3.at[0], isem).wait()

    def drain_scat():
        pltpu.make_async_copy(rows2.at[0], acc.at[didx3.at[0]], ssem).wait()

    def wait_gather(g):
        pltpu.make_async_copy(g.at[sidx3.at[0]], rows2.at[0], gsem).wait()

    def fire_scatter(ki, kr):
        pltpu.async_copy(rows2.at[kr], acc.at[didx3.at[ki]], ssem, add=True)

    for d in range(2):
        fetch_idx(d, 0, 0)

        def chunk(k, _, d=d):
            ki = lax.rem(k, 3)
            kr = lax.rem(k, 3)

            @pl.when(k >= 3)
            def _():
                drain_scat()
            wait_idx()

            @pl.when(k < NCHUNK - 1)
            def _():
                fetch_idx(d, k + 1, lax.rem(k + 1, 3))

            @pl.when(c == 0)
            def _():
                pltpu.async_copy(gLo.at[sidx3.at[ki]], rows2.at[kr], gsem)

            @pl.when(c == 1)
            def _():
                pltpu.async_copy(gHi.at[sidx3.at[ki]], rows2.at[kr], gsem)

            @pl.when(k >= 1)
            def _():
                wait_gather(gLo)
                fire_scatter(lax.rem(k - 1, 3), lax.rem(k - 1, 3))
            return 0
        lax.fori_loop(0, NCHUNK, chunk, 0)
        wait_gather(gLo)
        fire_scatter(lax.rem(NCHUNK - 1, 3), lax.rem(NCHUNK - 1, 3))
        drain_scat()
        drain_scat()
        drain_scat()

    plsc.subcore_barrier()

    @pl.when(c == 0)
    def _():
        for i in range(4):
            pltpu.sync_copy(acc.at[pl.ds(base + i * 1568, 1568)],
                            outLo.at[pl.ds(base + i * 1568, 1568)])

    @pl.when(c == 1)
    def _():
        for i in range(4):
            pltpu.sync_copy(acc.at[pl.ds(base + i * 1568, 1568)],
                            outHi.at[pl.ds(base + i * 1568, 1568)])


_prop16 = pl.kernel(
    _prop16_body,
    out_type=(jax.ShapeDtypeStruct((NPAD, F), jnp.float32),
              jax.ShapeDtypeStruct((NPAD, F), jnp.float32)),
    mesh=_mesh,
    compiler_params=_sc_params,
    scratch_types=[
        pltpu.VMEM((3, CH), jnp.int32),
        pltpu.VMEM((3, CH), jnp.int32),
        pltpu.VMEM((3, CH, F), jnp.float32),
        pltpu.VMEM_SHARED((NPAD, F), jnp.float32),
        pltpu.SemaphoreType.DMA,
        pltpu.SemaphoreType.DMA,
        pltpu.SemaphoreType.DMA,
    ],
)


def _prop1_body(eiF, g2, zcol, tp, tpart, part, sidx2, dbuf2, vbuf, accbuf,
                tbuf, sem, isem):
    c = lax.axis_index("c")
    s = lax.axis_index("s")
    pltpu.sync_copy(zcol, part)

    def fetch_idx(k, kb):
        pltpu.async_copy(eiF.at[c, s, pl.ds(k * MB, MB)], sidx2.at[kb],
                         isem)
        pltpu.async_copy(eiF.at[1 - c, s, pl.ds(k * MB, MB)],
                         dbuf2.at[kb], isem)

    def wait_idx():
        pltpu.make_async_copy(eiF.at[0, s, pl.ds(0, MB)], sidx2.at[0],
                              isem).wait()
        pltpu.make_async_copy(eiF.at[0, s, pl.ds(0, MB)], dbuf2.at[0],
                              isem).wait()

    fetch_idx(0, 0)

    def chunk(k, _):
        kb = lax.rem(k, 2)
        wait_idx()

        @pl.when(k < NMB - 1)
        def _():
            fetch_idx(k + 1, 1 - kb)
        sidx = sidx2.at[kb]
        dbuf = dbuf2.at[kb]
        for m in range(MB):
            pltpu.async_copy(g2.at[sidx.at[m]], vbuf.at[m], sem)
        for m in range(MB):
            pltpu.make_async_copy(g2.at[sidx.at[0]], vbuf.at[0], sem).wait()

            def iloop(i, _, m=m):
                sl = pl.ds(i * 16, 16)
                plsc.addupdate_scatter(part, [dbuf2[kb, m, sl]],
                                       vbuf[m, sl])
                return 0
            lax.fori_loop(0, CH // 16, iloop, 0)
        return 0
    lax.fori_loop(0, NMB, chunk, 0)

    _reduce_tiles(part, tpart, tp, c, s, accbuf, tbuf)


_prop1 = pl.kernel(
    _prop1_body,
    out_type=(jax.ShapeDtypeStruct((NC, NPAD), jnp.float32),
              jax.ShapeDtypeStruct((NC, NT, NPAD), jnp.float32)),
    mesh=_mesh,
    compiler_params=_sc_params,
    scratch_types=[
        pltpu.VMEM((NPAD,), jnp.float32),
        pltpu.VMEM((2, MB, CH), jnp.int32),
        pltpu.VMEM((2, MB, CH), jnp.int32),
        pltpu.VMEM((MB, CH), jnp.float32),
        pltpu.VMEM((TILE_N,), jnp.float32),
        pltpu.VMEM((TILE_N,), jnp.float32),
        pltpu.SemaphoreType.DMA,
        pltpu.SemaphoreType.DMA,
    ],
)


def _dense_a_body(xT_r, w1_r, b1_r, w2_r, b2_r, w3_r, b3_r,
                  g1_r, gb1_r, g2_r, gb2_r, g3_r, gb3_r,
                  wc1_r, dp_r,
                  g0_r, g1o_r, dinv_r,
                  s1_r, s2_r, s3_r, acc_r, st_r):
    p = pl.program_id(0)
    b = pl.program_id(1)
    eps = 1e-5
    glob = b * BR + lax.broadcasted_iota(jnp.int32, (1, BR), 1)
    msk = glob < N

    @pl.when((p == 0) & (b == 0))
    def _():
        acc_r[...] = jnp.zeros((64, 2), jnp.float32)

    @pl.when(p == 0)
    def _():
        a1 = jnp.dot(w1_r[...], xT_r[...],
                     preferred_element_type=jnp.float32) + b1_r[...]
        s1_r[b] = a1
        a1m = jnp.where(msk, a1, 0.0)
        acc_r[0:8, 0:1] += jnp.sum(a1m, axis=1, keepdims=True)
        acc_r[0:8, 1:2] += jnp.sum(a1m * a1m, axis=1, keepdims=True)

    @pl.when((p == 1) & (b == 0))
    def _():
        m = acc_r[0:8, 0:1] / N
        v = acc_r[0:8, 1:2] / N - m * m
        st_r[0:8, 0:1] = m
        st_r[0:8, 1:2] = lax.rsqrt(v + eps)

    @pl.when(p == 1)
    def _():
        h1 = _lrelu(g1_r[...] * (s1_r[b] - st_r[0:8, 0:1])
                    * st_r[0:8, 1:2] + gb1_r[...])
        a2 = jnp.dot(w2_r[...], h1,
                     preferred_element_type=jnp.float32) + b2_r[...]
        s2_r[b] = a2
        a2m = jnp.where(msk, a2, 0.0)
        acc_r[8:24, 0:1] += jnp.sum(a2m, axis=1, keepdims=True)
        acc_r[8:24, 1:2] += jnp.sum(a2m * a2m, axis=1, keepdims=True)

    @pl.when((p == 2) & (b == 0))
    def _():
        m = acc_r[8:24, 0:1] / N
        v = acc_r[8:24, 1:2] / N - m * m
        st_r[8:24, 0:1] = m
        st_r[8:24, 1:2] = lax.rsqrt(v + eps)

    @pl.when(p == 2)
    def _():
        h2 = _lrelu(g2_r[...] * (s2_r[b] - st_r[8:24, 0:1])
                    * st_r[8:24, 1:2] + gb2_r[...])
        a3 = jnp.dot(w3_r[...], h2,
                     preferred_element_type=jnp.float32) + b3_r[...]
        s3_r[b] = a3
        a3m = jnp.where(msk, a3, 0.0)
        acc_r[24:56, 0:1] += jnp.sum(a3m, axis=1, keepdims=True)
        acc_r[24:56, 1:2] += jnp.sum(a3m * a3m, axis=1, keepdims=True)

    @pl.when((p == 3) & (b == 0))
    def _():
        m = acc_r[24:56, 0:1] / N
        v = acc_r[24:56, 1:2] / N - m * m
        st_r[24:56, 0:1] = m
        st_r[24:56, 1:2] = lax.rsqrt(v + eps)

    @pl.when(p == 3)
    def _():
        h3 = _lrelu(g3_r[...] * (s3_r[b] - st_r[24:56, 0:1])
                    * st_r[24:56, 1:2] + gb3_r[...])
        h1pT = jnp.dot(wc1_r[...], h3, preferred_element_type=jnp.float32)
        deg = dp_r[0:1, :] + dp_r[1:2, :] + 1.0
        dinv = lax.rsqrt(deg)
        gT = dinv * h1pT
        gblk = jnp.transpose(gT)
        g0_r[...] = gblk[:, 0:16]
        g1o_r[...] = gblk[:, 16:32]
        dinv_r[...] = dinv


def _dense_b_body(s0_r, s1_r, g0_r, g1_r,
                  dinv_r, bc1_r, wc2_r, out_r):
    dinv = jnp.transpose(dinv_r[...])
    y2 = jnp.zeros((BRB, 1), jnp.float32)
    for i, (s_r, g_r) in enumerate(((s0_r, g0_r), (s1_r, g1_r))):
        z = _lrelu(dinv * (s_r[...] + g_r[...])
                   + bc1_r[...][:, i * 16:(i + 1) * 16])
        y2 = y2 + jnp.dot(z, wc2_r[...][i * 16:(i + 1) * 16],
                          preferred_element_type=jnp.float32)
    out_r[...] = jnp.transpose(dinv * y2)


def _dense_c_body(tp_r, g2_r, dinv_r, bc2_r, out_r):
    t = tp_r[0:1, :] + tp_r[1:2, :] + g2_r[...]
    out_r[...] = jnp.transpose(dinv_r[...] * t + bc2_r[...])


def kernel(x, edge_index, fc1_W, fc1_b, fc2_W, fc2_b, fc3_W, fc3_b,
           bn1_g, bn1_b, bn2_g, bn2_b, bn3_g, bn3_b,
           conv1_W, conv1_b, conv2_W, conv2_b):
    f32 = jnp.float32
    eiF = edge_index.reshape(2, NT, NCHUNK, CH)

    zcol = jnp.zeros((NPAD,), f32)
    dp, _dpart = _deg(eiF, zcol)

    col = lambda v: v.reshape(-1, 1)
    full = lambda shape: pl.BlockSpec(shape, lambda p, b: (0,) * len(shape))
    g_spec = pl.BlockSpec((BR, F),
                          lambda p, b: (jnp.where(p == 3, b, 0), 0))
    lane_spec = pl.BlockSpec((2, BR),
                             lambda p, b: (0, jnp.where(p == 0, b, 0)))
    deg_spec = pl.BlockSpec((2, BR),
                            lambda p, b: (0, jnp.where(p == 3, b, 0)))

    xT = jnp.pad(x.T, ((0, 0), (0, NPAD - N)))
    g0, g1, dinv = pl.pallas_call(
        _dense_a_body,
        grid=(4, NB),
        in_specs=[
            lane_spec,
            full((8, 2)), full((8, 1)),
            full((16, 8)), full((16, 1)),
            full((32, 16)), full((32, 1)),
            full((8, 1)), full((8, 1)),
            full((16, 1)), full((16, 1)),
            full((32, 1)), full((32, 1)),
            full((32, 32)),
            deg_spec,
        ],
        out_specs=[g_spec, g_spec,
                   pl.BlockSpec((1, BR),
                                lambda p, b: (0, jnp.where(p == 3, b, 0)))],
        out_shape=(jax.ShapeDtypeStruct((NPAD, F), f32),
                   jax.ShapeDtypeStruct((NPAD, F), f32),
                   jax.ShapeDtypeStruct((1, NPAD), f32)),
        scratch_shapes=[
            pltpu.VMEM((NB, 8, BR), f32),
            pltpu.VMEM((NB, 16, BR), f32),
            pltpu.VMEM((NB, 32, BR), f32),
            pltpu.VMEM((64, 2), f32),
            pltpu.VMEM((64, 2), f32),
        ],
    )(xT, fc1_W, col(fc1_b), fc2_W, col(fc2_b), fc3_W, col(fc3_b),
      col(bn1_g), col(bn1_b), col(bn2_g), col(bn2_b), col(bn3_g),
      col(bn3_b), conv1_W, dp)

    zrows = jnp.zeros((TILE_N, F), f32)
    s0, s1 = _prop16(eiF, g0, g1, zrows)

    gb_spec = pl.BlockSpec((BRB, F), lambda b: (b, 0))
    row1_spec = pl.BlockSpec((1, BRB), lambda b: (0, b))
    fullb = lambda shape: pl.BlockSpec(shape, lambda b: (0,) * len(shape))

    g2 = pl.pallas_call(
        _dense_b_body,
        grid=(NBB,),
        in_specs=[gb_spec, gb_spec, gb_spec, gb_spec,
                  row1_spec, fullb((1, 32)), fullb((32, 1))],
        out_specs=row1_spec,
        out_shape=jax.ShapeDtypeStruct((1, NPAD), f32),
    )(s0, s1, g0, g1, dinv,
      conv1_b.reshape(1, 32), conv2_W.T)

    tp, _tpart = _prop1(eiF, g2.reshape(NPAD), zcol)

    out = pl.pallas_call(
        _dense_c_body,
        grid=(NBB,),
        in_specs=[pl.BlockSpec((2, BRB), lambda b: (0, b)),
                  row1_spec, row1_spec, fullb((1, 1))],
        out_specs=pl.BlockSpec((BRB, 1), lambda b: (b, 0)),
        out_shape=jax.ShapeDtypeStruct((NPAD, 1), f32),
    )(tp, g2, dinv, conv2_b.reshape(1, 1))
    return out[:N]
